# Initial kernel scaffold; baseline (speedup 1.0000x reference)
import jax
import jax.numpy as jnp
from jax.experimental import pallas as pl


def kernel(x, edge_index, edge_attr, y, adj, W1, a_src1, a_dst1, a_edge1, Wp1, a_src_p1, a_dst_p1, a_edge_p1, Wp2, a_src_p2, a_dst_p2, a_edge_p2, Wf1, bf1, Wf2, bf2):
    raise NotImplementedError("write your pallas kernel here")


# speedup vs baseline: 3.1948x; 3.1948x over previous
import functools

import jax
import jax.numpy as jnp
from jax import lax
from jax.experimental import pallas as pl
from jax.experimental.pallas import tpu as pltpu
from jax.experimental.pallas import tpu_sc as plsc

N_PAD = 10240
E_PAD = 163840
BN = 1024
BE = 2048
ADJ_BR = 400
ADJ_BC = 1024


def _node_prep_body(x_ref, w_ref, asrc_ref, adst_ref, h_ref, ss_ref, sd_ref):
    h = jnp.dot(x_ref[...], w_ref[...], preferred_element_type=jnp.float32)
    h_ref[...] = h
    ss_ref[...] = jnp.dot(h, asrc_ref[...], preferred_element_type=jnp.float32)
    sd_ref[...] = jnp.dot(h, adst_ref[...], preferred_element_type=jnp.float32)


def _node_prep(xp, w, asrc, adst):
    npad, din = xp.shape
    dh = w.shape[1]
    grid = npad // BN
    return pl.pallas_call(
        _node_prep_body,
        grid=(grid,),
        in_specs=[
            pl.BlockSpec((BN, din), lambda i: (i, 0)),
            pl.BlockSpec((din, dh), lambda i: (0, 0)),
            pl.BlockSpec((dh, 16), lambda i: (0, 0)),
            pl.BlockSpec((dh, 16), lambda i: (0, 0)),
        ],
        out_specs=[
            pl.BlockSpec((BN, dh), lambda i: (i, 0)),
            pl.BlockSpec((BN, 16), lambda i: (i, 0)),
            pl.BlockSpec((BN, 16), lambda i: (i, 0)),
        ],
        out_shape=[
            jax.ShapeDtypeStruct((npad, dh), jnp.float32),
            jax.ShapeDtypeStruct((npad, 16), jnp.float32),
            jax.ShapeDtypeStruct((npad, 16), jnp.float32),
        ],
    )(xp, w, asrc, adst)


def _edge_prep_body(ea_ref, ae1_ref, ae2_ref, et1_ref, et2_ref):
    ea = ea_ref[...]
    et1_ref[...] = jnp.dot(ea, ae1_ref[...], preferred_element_type=jnp.float32)
    et2_ref[...] = jnp.dot(ea, ae2_ref[...], preferred_element_type=jnp.float32)


def _edge_prep(eap, ae1, ae2):
    epad, de = eap.shape
    grid = epad // BE
    return pl.pallas_call(
        _edge_prep_body,
        grid=(grid,),
        in_specs=[
            pl.BlockSpec((BE, de), lambda i: (i, 0)),
            pl.BlockSpec((de, 16), lambda i: (0, 0)),
            pl.BlockSpec((de, 16), lambda i: (0, 0)),
        ],
        out_specs=[
            pl.BlockSpec((BE, 16), lambda i: (i, 0)),
            pl.BlockSpec((BE, 16), lambda i: (i, 0)),
        ],
        out_shape=[
            jax.ShapeDtypeStruct((epad, 16), jnp.float32),
            jax.ShapeDtypeStruct((epad, 16), jnp.float32),
        ],
    )(eap, ae1, ae2)


def _maxes_body(ss_ref, sd_ref, et_ref, k_ref, a1_ref, a2_ref, a3_ref, *, ng):
    i = pl.program_id(0)

    @pl.when(i == 0)
    def _():
        a1_ref[...] = jnp.full_like(a1_ref, -1e30)
        a2_ref[...] = jnp.full_like(a2_ref, -1e30)
        a3_ref[...] = jnp.full_like(a3_ref, -1e30)

    def colmax(r):
        return jnp.broadcast_to(jnp.max(r[...], axis=0)[None, :], (8, 16))

    a1_ref[...] = jnp.maximum(a1_ref[...], colmax(ss_ref))
    a2_ref[...] = jnp.maximum(a2_ref[...], colmax(sd_ref))
    a3_ref[...] = jnp.maximum(a3_ref[...], colmax(et_ref))

    @pl.when(i == ng - 1)
    def _():
        m = a1_ref[...] + a2_ref[...] + a3_ref[...]
        k_ref[...] = jnp.maximum(m, 0.2 * m)


def _maxes(ss, sd, et):
    npad = ss.shape[0]
    epad = et.shape[0]
    nb = npad // BN
    ng = epad // BE
    return pl.pallas_call(
        functools.partial(_maxes_body, ng=ng),
        grid=(ng,),
        in_specs=[
            pl.BlockSpec((BN, 16), lambda i: (i % nb, 0)),
            pl.BlockSpec((BN, 16), lambda i: (i % nb, 0)),
            pl.BlockSpec((BE, 16), lambda i: (i, 0)),
        ],
        out_specs=pl.BlockSpec((8, 16), lambda i: (0, 0)),
        out_shape=jax.ShapeDtypeStruct((8, 16), jnp.float32),
        scratch_shapes=[
            pltpu.VMEM((8, 16), jnp.float32),
            pltpu.VMEM((8, 16), jnp.float32),
            pltpu.VMEM((8, 16), jnp.float32),
        ],
    )(ss, sd, et)


def _rden_body(d_ref, r_ref):
    d = d_ref[0] + d_ref[1]
    r_ref[...] = 1.0 / (d + 1e-16)


def _rden(dparts):
    npad = dparts.shape[1]
    return pl.pallas_call(
        _rden_body,
        out_shape=jax.ShapeDtypeStruct((npad, 16), jnp.float32),
    )(dparts)


def _prep2_body(a_ref, w_ref, asrc_ref, adst_ref, x1_ref, h2_ref, ss_ref, sd_ref):
    x1 = a_ref[0] + a_ref[1]
    x1_ref[...] = x1
    h2 = jnp.dot(x1, w_ref[...], preferred_element_type=jnp.float32)
    h2_ref[...] = h2
    ss_ref[...] = jnp.dot(h2, asrc_ref[...], preferred_element_type=jnp.float32)
    sd_ref[...] = jnp.dot(h2, adst_ref[...], preferred_element_type=jnp.float32)


def _prep2(aggparts, wp, asrc, adst):
    npad = aggparts.shape[1]
    grid = npad // BN
    return pl.pallas_call(
        _prep2_body,
        grid=(grid,),
        in_specs=[
            pl.BlockSpec((2, BN, 32), lambda i: (0, i, 0)),
            pl.BlockSpec((32, 32), lambda i: (0, 0)),
            pl.BlockSpec((32, 16), lambda i: (0, 0)),
            pl.BlockSpec((32, 16), lambda i: (0, 0)),
        ],
        out_specs=[
            pl.BlockSpec((BN, 32), lambda i: (i, 0)),
            pl.BlockSpec((BN, 32), lambda i: (i, 0)),
            pl.BlockSpec((BN, 16), lambda i: (i, 0)),
            pl.BlockSpec((BN, 16), lambda i: (i, 0)),
        ],
        out_shape=[
            jax.ShapeDtypeStruct((npad, 32), jnp.float32),
            jax.ShapeDtypeStruct((npad, 32), jnp.float32),
            jax.ShapeDtypeStruct((npad, 16), jnp.float32),
            jax.ShapeDtypeStruct((npad, 16), jnp.float32),
        ],
    )(aggparts, wp, asrc, adst)


def _smax_body(s1_ref, s_ref, ent_ref, *, nreal):
    i = pl.program_id(0)
    z = s1_ref[...]
    m = jnp.max(z, axis=1, keepdims=True)
    e = jnp.exp(z - m)
    sm = e / jnp.sum(e, axis=1, keepdims=True)
    rid = i * BN + lax.broadcasted_iota(jnp.int32, sm.shape, 0)
    sm = jnp.where(rid < nreal, sm, 0.0)
    s_ref[...] = sm
    ent = -jnp.sum(sm * jnp.log(sm + 1e-15))

    @pl.when(i == 0)
    def _():
        ent_ref[0, 0] = 0.0

    ent_ref[0, 0] += ent


def _smax(s1, nreal):
    npad = s1.shape[0]
    grid = npad // BN
    return pl.pallas_call(
        functools.partial(_smax_body, nreal=nreal),
        grid=(grid,),
        in_specs=[pl.BlockSpec((BN, 32), lambda i: (i, 0))],
        out_specs=[
            pl.BlockSpec((BN, 32), lambda i: (i, 0)),
            pl.BlockSpec((1, 1), lambda i: (0, 0),
                         memory_space=pltpu.SMEM),
        ],
        out_shape=[
            jax.ShapeDtypeStruct((npad, 32), jnp.float32),
            jax.ShapeDtypeStruct((1, 1), jnp.float32),
        ],
    )(s1)


def _adj_body(adj_ref, sk_ref, si_ref, x1_ref, adjp_ref, xp_ref, sts_ref,
              fro_ref, tmp_ref, *, nk, ncols):
    i = pl.program_id(0)
    k = pl.program_id(1)

    blk = adj_ref[...]
    colid = k * ADJ_BC + lax.broadcasted_iota(jnp.int32, blk.shape, 1)
    blk = jnp.where(colid < ncols, blk, 0.0)

    @pl.when(jnp.logical_and(i == 0, k == 0))
    def _():
        adjp_ref[...] = jnp.zeros_like(adjp_ref)
        xp_ref[...] = jnp.zeros_like(xp_ref)
        sts_ref[...] = jnp.zeros_like(sts_ref)
        fro_ref[0, 0] = 0.0

    fro_ref[0, 0] += jnp.sum(blk * blk)

    part = jnp.dot(blk, sk_ref[...], preferred_element_type=jnp.float32)

    @pl.when(k == 0)
    def _():
        tmp_ref[...] = part
        si = si_ref[...]
        xp_ref[...] += lax.dot_general(
            si, x1_ref[...], (((0,), (0,)), ((), ())),
            preferred_element_type=jnp.float32)
        sts_ref[...] += lax.dot_general(
            si, si, (((0,), (0,)), ((), ())),
            preferred_element_type=jnp.float32)

    @pl.when(k > 0)
    def _():
        tmp_ref[...] += part

    @pl.when(k == nk - 1)
    def _():
        adjp_ref[...] += lax.dot_general(
            si_ref[...], tmp_ref[...], (((0,), (0,)), ((), ())),
            preferred_element_type=jnp.float32)


def _adj_pass(adj, s, x1):
    nrows, ncols = adj.shape
    ni = nrows // ADJ_BR
    nk = (ncols + ADJ_BC - 1) // ADJ_BC
    return pl.pallas_call(
        functools.partial(_adj_body, nk=nk, ncols=ncols),
        grid=(ni, nk),
        in_specs=[
            pl.BlockSpec((ADJ_BR, ADJ_BC), lambda i, k: (i, k)),
            pl.BlockSpec((ADJ_BC, 32), lambda i, k: (k, 0)),
            pl.BlockSpec((ADJ_BR, 32), lambda i, k: (i, 0)),
            pl.BlockSpec((ADJ_BR, 32), lambda i, k: (i, 0)),
        ],
        out_specs=[
            pl.BlockSpec((32, 32), lambda i, k: (0, 0)),
            pl.BlockSpec((32, 32), lambda i, k: (0, 0)),
            pl.BlockSpec((32, 32), lambda i, k: (0, 0)),
            pl.BlockSpec((1, 1), lambda i, k: (0, 0),
                         memory_space=pltpu.SMEM),
        ],
        out_shape=[
            jax.ShapeDtypeStruct((32, 32), jnp.float32),
            jax.ShapeDtypeStruct((32, 32), jnp.float32),
            jax.ShapeDtypeStruct((32, 32), jnp.float32),
            jax.ShapeDtypeStruct((1, 1), jnp.float32),
        ],
        scratch_shapes=[pltpu.VMEM((ADJ_BR, 32), jnp.float32)],
        compiler_params=pltpu.CompilerParams(
            dimension_semantics=("arbitrary", "arbitrary")),
    )(adj, s, s, x1)


def _tail_body(adjp_ref, xp_ref, sts_ref, fro_ref, ent_ref, wp2_ref, asp2_ref,
               adp2_ref, aep2_ref, wf1_ref, bf1_ref, wf2_ref, bf2_ref,
               outa_ref, outb_ref):
    adjp = adjp_ref[...]
    x2 = xp_ref[...]
    h3 = jnp.dot(x2, wp2_ref[...], preferred_element_type=jnp.float32)
    ss3 = jnp.dot(h3, asp2_ref[...], preferred_element_type=jnp.float32)
    sd3m = lax.dot_general(adp2_ref[...], h3, (((0,), (1,)), ((), ())),
                           preferred_element_type=jnp.float32)
    alpha = ss3[:, 0:1] + sd3m[0:1, :] + adjp * aep2_ref[0, 0]
    alpha = jnp.maximum(alpha, 0.2 * alpha)
    cmax = jnp.max(alpha, axis=0, keepdims=True)
    ex = jnp.exp(alpha - cmax)
    att = ex / (jnp.sum(ex, axis=0, keepdims=True) + 1e-16)
    s2 = lax.dot_general(att, h3, (((0,), (0,)), ((), ())),
                         preferred_element_type=jnp.float32)

    colmask = lax.broadcasted_iota(jnp.int32, s2.shape, 1) < 4
    z = jnp.where(colmask, s2, -1e30)
    m2 = jnp.max(z, axis=1, keepdims=True)
    e2 = jnp.where(colmask, jnp.exp(z - m2), 0.0)
    s2s = e2 / jnp.sum(e2, axis=1, keepdims=True)
    ent2 = -jnp.sum(s2s * jnp.log(s2s + 1e-15)) / 32.0

    x3 = lax.dot_general(s2s, x2, (((0,), (0,)), ((), ())),
                         preferred_element_type=jnp.float32)
    adjs2 = jnp.dot(adjp, s2s, preferred_element_type=jnp.float32)
    adjp2 = lax.dot_general(s2s, adjs2, (((0,), (0,)), ((), ())),
                            preferred_element_type=jnp.float32)
    sts2 = lax.dot_general(s2s, s2s, (((0,), (0,)), ((), ())),
                           preferred_element_type=jnp.float32)
    eye8 = (lax.broadcasted_iota(jnp.int32, (8, 8), 0)
            == lax.broadcasted_iota(jnp.int32, (8, 8), 1))
    tr2 = jnp.sum(jnp.where(eye8, adjp2, 0.0))
    link2sq = jnp.sum(adjp * adjp) - 2.0 * tr2 + jnp.sum(sts2 * sts2)
    link2 = jnp.sqrt(jnp.maximum(link2sq, 1e-12)) / 32.0
    reg2 = link2 + ent2

    sts1 = sts_ref[...]
    eye32 = (lax.broadcasted_iota(jnp.int32, (32, 32), 0)
             == lax.broadcasted_iota(jnp.int32, (32, 32), 1))
    tr1 = jnp.sum(jnp.where(eye32, adjp, 0.0))
    link1sq = fro_ref[0, 0] - 2.0 * tr1 + jnp.sum(sts1 * sts1)
    link1 = jnp.sqrt(jnp.maximum(link1sq, 1e-12)) / 10000.0
    ent1 = ent_ref[0, 0] / 10000.0
    reg = (link1 + ent1) * 10.0 + reg2 * 0.1

    acc = bf1_ref[...]
    for r in range(8):
        acc = acc + jnp.dot(x3[r:r + 1, :], wf1_ref[r],
                            preferred_element_type=jnp.float32)
    h1f = jnp.maximum(acc, 0.0)
    out2 = jnp.dot(h1f, wf2_ref[...], preferred_element_type=jnp.float32) \
        + bf2_ref[...]
    outa_ref[...] = jnp.broadcast_to(out2, (8, 128))
    outb_ref[...] = jnp.full((8, 128), reg)


def _tail(adjp, xp, sts, fro, ent, wp2, asp2, adp2, aep2, wf1g, bf1p, wf2p,
          bf2p):
    vm = pl.BlockSpec(memory_space=pltpu.VMEM)
    sm = pl.BlockSpec(memory_space=pltpu.SMEM)
    return pl.pallas_call(
        _tail_body,
        in_specs=[vm, vm, vm, sm, sm, vm, vm, vm, sm, vm, vm, vm, vm],
        out_shape=[
            jax.ShapeDtypeStruct((8, 128), jnp.float32),
            jax.ShapeDtypeStruct((8, 128), jnp.float32),
        ],
    )(adjp, xp, sts, fro, ent, wp2, asp2, adp2, aep2, wf1g, bf1p, wf2p, bf2p)


def _conv_segops(ss, sd, et, k16, h, srcp, dstp, headmap):
    npad = ss.shape[0]
    alpha = ss[srcp] + sd[dstp] + et
    alpha = jnp.maximum(alpha, 0.2 * alpha) - k16[None, :]
    ex = jnp.exp(alpha)
    den = jax.ops.segment_sum(ex, dstp, num_segments=npad)
    rden = 1.0 / (den + 1e-16)
    att = ex * rden[dstp]
    msg = att[:, headmap] * h[srcp]
    agg = jax.ops.segment_sum(msg, dstp, num_segments=npad)
    return jnp.stack([agg, jnp.zeros_like(agg)])


def kernel(x, edge_index, edge_attr, y, adj, W1, a_src1, a_dst1, a_edge1,
           Wp1, a_src_p1, a_dst_p1, a_edge_p1, Wp2, a_src_p2, a_dst_p2,
           a_edge_p2, Wf1, bf1, Wf2, bf2):
    n, dfeat = x.shape
    e = edge_index.shape[1]

    xp_in = jnp.pad(x, ((0, N_PAD - n), (0, 0)))
    srcp = jnp.concatenate(
        [edge_index[0].astype(jnp.int32),
         jnp.zeros((E_PAD - e,), jnp.int32)])
    dstp = jnp.concatenate(
        [edge_index[1].astype(jnp.int32),
         jnp.full((E_PAD - e,), n, jnp.int32)])
    eap = jnp.pad(edge_attr, ((0, E_PAD - e), (0, 0)))

    w1p = jnp.pad(W1, ((0, 0), (0, 2)))
    asrc1 = jnp.zeros((32, 16), jnp.float32)
    adst1 = jnp.zeros((32, 16), jnp.float32)
    for hh in range(5):
        asrc1 = asrc1.at[hh * 6:(hh + 1) * 6, hh].set(a_src1[hh])
        adst1 = adst1.at[hh * 6:(hh + 1) * 6, hh].set(a_dst1[hh])
    ae1p = jnp.pad(a_edge1, ((0, 0), (0, 11)))
    ae2p = jnp.pad(a_edge_p1, ((0, 0), (0, 15)))

    wp1p = jnp.pad(Wp1, ((0, 2), (0, 0)))
    asrc2 = jnp.zeros((32, 16), jnp.float32).at[:, 0].set(a_src_p1[0])
    adst2 = jnp.zeros((32, 16), jnp.float32).at[:, 0].set(a_dst_p1[0])

    headmap1 = jnp.array([0] * 6 + [1] * 6 + [2] * 6 + [3] * 6 + [4] * 6
                         + [7, 7], jnp.int32)
    headmap2 = jnp.zeros((32,), jnp.int32)

    h1p, ss1, sd1 = _node_prep(xp_in, w1p, asrc1, adst1)
    et1, et2 = _edge_prep(eap, ae1p, ae2p)
    k1 = _maxes(ss1, sd1, et1)[0]

    aggparts1 = _conv_segops(ss1, sd1, et1, k1, h1p, srcp, dstp, headmap1)

    x1, h2p, ss2, sd2 = _prep2(aggparts1, wp1p, asrc2, adst2)
    k2 = _maxes(ss2, sd2, et2)[0]

    aggparts2 = _conv_segops(ss2, sd2, et2, k2, h2p, srcp, dstp, headmap2)
    s1_logits = aggparts2[0] + aggparts2[1]

    s, ent_sum = _smax(s1_logits, n)
    adjp, xp, sts, fro = _adj_pass(adj, s, x1)

    wp2p = jnp.pad(Wp2, ((0, 2), (0, 4)))
    asp2 = jnp.zeros((8, 8), jnp.float32).at[0:4, 0].set(a_src_p2[0])
    adp2 = jnp.zeros((8, 8), jnp.float32).at[0:4, 0].set(a_dst_p2[0])
    aep2 = a_edge_p2.reshape(1, 1)
    wf1g = jnp.zeros((8, 32, 32), jnp.float32).at[0:4, 0:30, :].set(
        Wf1.reshape(4, 30, 32))
    bf1p = bf1.reshape(1, 32)
    wf2p = jnp.zeros((32, 128), jnp.float32).at[:, 0:2].set(Wf2)
    bf2p = jnp.zeros((1, 128), jnp.float32).at[0, 0:2].set(bf2)

    outa, outb = _tail(adjp, xp, sts, fro, ent_sum, wp2p, asp2, adp2, aep2,
                       wf1g, bf1p, wf2p, bf2p)
    return outa[0:1, 0:2], outb[0, 0]


# speedup vs baseline: 13.5344x; 4.2364x over previous
import functools

import jax
import jax.numpy as jnp
from jax import lax
from jax.experimental import pallas as pl
from jax.experimental.pallas import tpu as pltpu
from jax.experimental.pallas import tpu_sc as plsc

N_PAD = 10240
E_PAD = 163840
BN = 1024
BE = 2048
ADJ_BR = 400
ADJ_BC = 1024


def _node_prep_body(x_ref, w_ref, asrc_ref, adst_ref, h_ref, ss_ref, sd_ref):
    h = jnp.dot(x_ref[...], w_ref[...], preferred_element_type=jnp.float32)
    h_ref[...] = h
    ss_ref[...] = jnp.dot(h, asrc_ref[...], preferred_element_type=jnp.float32)
    sd_ref[...] = jnp.dot(h, adst_ref[...], preferred_element_type=jnp.float32)


def _node_prep(xp, w, asrc, adst):
    npad, din = xp.shape
    dh = w.shape[1]
    grid = npad // BN
    return pl.pallas_call(
        _node_prep_body,
        grid=(grid,),
        in_specs=[
            pl.BlockSpec((BN, din), lambda i: (i, 0)),
            pl.BlockSpec((din, dh), lambda i: (0, 0)),
            pl.BlockSpec((dh, 32), lambda i: (0, 0)),
            pl.BlockSpec((dh, 32), lambda i: (0, 0)),
        ],
        out_specs=[
            pl.BlockSpec((BN, dh), lambda i: (i, 0)),
            pl.BlockSpec((BN, 32), lambda i: (i, 0)),
            pl.BlockSpec((BN, 32), lambda i: (i, 0)),
        ],
        out_shape=[
            jax.ShapeDtypeStruct((npad, dh), jnp.float32),
            jax.ShapeDtypeStruct((npad, 32), jnp.float32),
            jax.ShapeDtypeStruct((npad, 32), jnp.float32),
        ],
    )(xp, w, asrc, adst)


def _edge_prep_body(ea_ref, ae1_ref, ae2_ref, et1_ref, et2_ref):
    ea = ea_ref[...]
    et1_ref[...] = jnp.dot(ea, ae1_ref[...], preferred_element_type=jnp.float32)
    et2_ref[...] = jnp.dot(ea, ae2_ref[...], preferred_element_type=jnp.float32)


def _edge_prep(eap, ae1, ae2):
    epad, de = eap.shape
    grid = epad // BE
    return pl.pallas_call(
        _edge_prep_body,
        grid=(grid,),
        in_specs=[
            pl.BlockSpec((BE, de), lambda i: (i, 0)),
            pl.BlockSpec((de, 32), lambda i: (0, 0)),
            pl.BlockSpec((de, 32), lambda i: (0, 0)),
        ],
        out_specs=[
            pl.BlockSpec((BE, 32), lambda i: (i, 0)),
            pl.BlockSpec((BE, 32), lambda i: (i, 0)),
        ],
        out_shape=[
            jax.ShapeDtypeStruct((epad, 32), jnp.float32),
            jax.ShapeDtypeStruct((epad, 32), jnp.float32),
        ],
    )(eap, ae1, ae2)


def _maxes_body(ss_ref, sd_ref, et_ref, k_ref, a1_ref, a2_ref, a3_ref, *, ng):
    i = pl.program_id(0)

    @pl.when(i == 0)
    def _():
        a1_ref[...] = jnp.full_like(a1_ref, -1e30)
        a2_ref[...] = jnp.full_like(a2_ref, -1e30)
        a3_ref[...] = jnp.full_like(a3_ref, -1e30)

    def colmax(r):
        return jnp.broadcast_to(jnp.max(r[...], axis=0)[None, :], (8, 32))

    a1_ref[...] = jnp.maximum(a1_ref[...], colmax(ss_ref))
    a2_ref[...] = jnp.maximum(a2_ref[...], colmax(sd_ref))
    a3_ref[...] = jnp.maximum(a3_ref[...], colmax(et_ref))

    @pl.when(i == ng - 1)
    def _():
        m = a1_ref[...] + a2_ref[...] + a3_ref[...]
        k_ref[...] = jnp.maximum(m, 0.2 * m)


def _maxes(ss, sd, et):
    npad = ss.shape[0]
    epad = et.shape[0]
    nb = npad // BN
    ng = epad // BE
    return pl.pallas_call(
        functools.partial(_maxes_body, ng=ng),
        grid=(ng,),
        in_specs=[
            pl.BlockSpec((BN, 32), lambda i: (i % nb, 0)),
            pl.BlockSpec((BN, 32), lambda i: (i % nb, 0)),
            pl.BlockSpec((BE, 32), lambda i: (i, 0)),
        ],
        out_specs=pl.BlockSpec((8, 32), lambda i: (0, 0)),
        out_shape=jax.ShapeDtypeStruct((8, 32), jnp.float32),
        scratch_shapes=[
            pltpu.VMEM((8, 32), jnp.float32),
            pltpu.VMEM((8, 32), jnp.float32),
            pltpu.VMEM((8, 32), jnp.float32),
        ],
    )(ss, sd, et)


def _rden_body(d_ref, r_ref):
    d = d_ref[0] + d_ref[1]
    r_ref[...] = 1.0 / (d + 1e-16)


def _rden(dparts):
    npad = dparts.shape[1]
    return pl.pallas_call(
        _rden_body,
        out_shape=jax.ShapeDtypeStruct((npad, 32), jnp.float32),
    )(dparts)


def _prep2_body(a_ref, w_ref, asrc_ref, adst_ref, x1_ref, h2_ref, ss_ref,
                sd_ref):
    x1 = a_ref[0] + a_ref[1]
    x1_ref[...] = x1
    h2 = jnp.dot(x1, w_ref[...], preferred_element_type=jnp.float32)
    h2_ref[...] = h2
    ss_ref[...] = jnp.dot(h2, asrc_ref[...], preferred_element_type=jnp.float32)
    sd_ref[...] = jnp.dot(h2, adst_ref[...], preferred_element_type=jnp.float32)


def _prep2(aggparts, wp, asrc, adst):
    npad = aggparts.shape[1]
    grid = npad // BN
    return pl.pallas_call(
        _prep2_body,
        grid=(grid,),
        in_specs=[
            pl.BlockSpec((2, BN, 32), lambda i: (0, i, 0)),
            pl.BlockSpec((32, 32), lambda i: (0, 0)),
            pl.BlockSpec((32, 32), lambda i: (0, 0)),
            pl.BlockSpec((32, 32), lambda i: (0, 0)),
        ],
        out_specs=[
            pl.BlockSpec((BN, 32), lambda i: (i, 0)),
            pl.BlockSpec((BN, 32), lambda i: (i, 0)),
            pl.BlockSpec((BN, 32), lambda i: (i, 0)),
            pl.BlockSpec((BN, 32), lambda i: (i, 0)),
        ],
        out_shape=[
            jax.ShapeDtypeStruct((npad, 32), jnp.float32),
            jax.ShapeDtypeStruct((npad, 32), jnp.float32),
            jax.ShapeDtypeStruct((npad, 32), jnp.float32),
            jax.ShapeDtypeStruct((npad, 32), jnp.float32),
        ],
    )(aggparts, wp, asrc, adst)


def _smax_body(s1_ref, s_ref, ent_ref, *, nreal):
    i = pl.program_id(0)
    z = s1_ref[0] + s1_ref[1]
    m = jnp.max(z, axis=1, keepdims=True)
    e = jnp.exp(z - m)
    sm = e / jnp.sum(e, axis=1, keepdims=True)
    rid = i * BN + lax.broadcasted_iota(jnp.int32, sm.shape, 0)
    sm = jnp.where(rid < nreal, sm, 0.0)
    s_ref[...] = sm
    ent = -jnp.sum(sm * jnp.log(sm + 1e-15))

    @pl.when(i == 0)
    def _():
        ent_ref[0, 0] = 0.0

    ent_ref[0, 0] += ent


def _smax(s1parts, nreal):
    npad = s1parts.shape[1]
    grid = npad // BN
    return pl.pallas_call(
        functools.partial(_smax_body, nreal=nreal),
        grid=(grid,),
        in_specs=[pl.BlockSpec((2, BN, 32), lambda i: (0, i, 0))],
        out_specs=[
            pl.BlockSpec((BN, 32), lambda i: (i, 0)),
            pl.BlockSpec((1, 1), lambda i: (0, 0),
                         memory_space=pltpu.SMEM),
        ],
        out_shape=[
            jax.ShapeDtypeStruct((npad, 32), jnp.float32),
            jax.ShapeDtypeStruct((1, 1), jnp.float32),
        ],
    )(s1parts)


def _adj_body(adj_ref, sk_ref, si_ref, x1_ref, adjp_ref, xp_ref, sts_ref,
              fro_ref, tmp_ref, *, nk, ncols):
    i = pl.program_id(0)
    k = pl.program_id(1)

    blk = adj_ref[...]
    colid = k * ADJ_BC + lax.broadcasted_iota(jnp.int32, blk.shape, 1)
    blk = jnp.where(colid < ncols, blk, 0.0)

    @pl.when(jnp.logical_and(i == 0, k == 0))
    def _():
        adjp_ref[...] = jnp.zeros_like(adjp_ref)
        xp_ref[...] = jnp.zeros_like(xp_ref)
        sts_ref[...] = jnp.zeros_like(sts_ref)
        fro_ref[0, 0] = 0.0

    fro_ref[0, 0] += jnp.sum(blk * blk)

    part = jnp.dot(blk, sk_ref[...], preferred_element_type=jnp.float32)

    @pl.when(k == 0)
    def _():
        tmp_ref[...] = part
        si = si_ref[...]
        xp_ref[...] += lax.dot_general(
            si, x1_ref[...], (((0,), (0,)), ((), ())),
            preferred_element_type=jnp.float32)
        sts_ref[...] += lax.dot_general(
            si, si, (((0,), (0,)), ((), ())),
            preferred_element_type=jnp.float32)

    @pl.when(k > 0)
    def _():
        tmp_ref[...] += part

    @pl.when(k == nk - 1)
    def _():
        adjp_ref[...] += lax.dot_general(
            si_ref[...], tmp_ref[...], (((0,), (0,)), ((), ())),
            preferred_element_type=jnp.float32)


def _adj_pass(adj, s, x1):
    nrows, ncols = adj.shape
    ni = nrows // ADJ_BR
    nk = (ncols + ADJ_BC - 1) // ADJ_BC
    return pl.pallas_call(
        functools.partial(_adj_body, nk=nk, ncols=ncols),
        grid=(ni, nk),
        in_specs=[
            pl.BlockSpec((ADJ_BR, ADJ_BC), lambda i, k: (i, k)),
            pl.BlockSpec((ADJ_BC, 32), lambda i, k: (k, 0)),
            pl.BlockSpec((ADJ_BR, 32), lambda i, k: (i, 0)),
            pl.BlockSpec((ADJ_BR, 32), lambda i, k: (i, 0)),
        ],
        out_specs=[
            pl.BlockSpec((32, 32), lambda i, k: (0, 0)),
            pl.BlockSpec((32, 32), lambda i, k: (0, 0)),
            pl.BlockSpec((32, 32), lambda i, k: (0, 0)),
            pl.BlockSpec((1, 1), lambda i, k: (0, 0),
                         memory_space=pltpu.SMEM),
        ],
        out_shape=[
            jax.ShapeDtypeStruct((32, 32), jnp.float32),
            jax.ShapeDtypeStruct((32, 32), jnp.float32),
            jax.ShapeDtypeStruct((32, 32), jnp.float32),
            jax.ShapeDtypeStruct((1, 1), jnp.float32),
        ],
        scratch_shapes=[pltpu.VMEM((ADJ_BR, 32), jnp.float32)],
        compiler_params=pltpu.CompilerParams(
            dimension_semantics=("arbitrary", "arbitrary")),
    )(adj, s, s, x1)


def _tail_body(adjp_ref, xp_ref, sts_ref, fro_ref, ent_ref, wp2_ref, asp2_ref,
               adp2_ref, aep2_ref, wf1_ref, bf1_ref, wf2_ref, bf2_ref,
               outa_ref, outb_ref):
    adjp = adjp_ref[...]
    x2 = xp_ref[...]
    h3 = jnp.dot(x2, wp2_ref[...], preferred_element_type=jnp.float32)
    ss3 = jnp.dot(h3, asp2_ref[...], preferred_element_type=jnp.float32)
    sd3m = lax.dot_general(adp2_ref[...], h3, (((0,), (1,)), ((), ())),
                           preferred_element_type=jnp.float32)
    alpha = ss3[:, 0:1] + sd3m[0:1, :] + adjp * aep2_ref[0, 0]
    alpha = jnp.maximum(alpha, 0.2 * alpha)
    cmax = jnp.max(alpha, axis=0, keepdims=True)
    ex = jnp.exp(alpha - cmax)
    att = ex / (jnp.sum(ex, axis=0, keepdims=True) + 1e-16)
    s2 = lax.dot_general(att, h3, (((0,), (0,)), ((), ())),
                         preferred_element_type=jnp.float32)

    colmask = lax.broadcasted_iota(jnp.int32, s2.shape, 1) < 4
    z = jnp.where(colmask, s2, -1e30)
    m2 = jnp.max(z, axis=1, keepdims=True)
    e2 = jnp.where(colmask, jnp.exp(z - m2), 0.0)
    s2s = e2 / jnp.sum(e2, axis=1, keepdims=True)
    ent2 = -jnp.sum(s2s * jnp.log(s2s + 1e-15)) / 32.0

    x3 = lax.dot_general(s2s, x2, (((0,), (0,)), ((), ())),
                         preferred_element_type=jnp.float32)
    adjs2 = jnp.dot(adjp, s2s, preferred_element_type=jnp.float32)
    adjp2 = lax.dot_general(s2s, adjs2, (((0,), (0,)), ((), ())),
                            preferred_element_type=jnp.float32)
    sts2 = lax.dot_general(s2s, s2s, (((0,), (0,)), ((), ())),
                           preferred_element_type=jnp.float32)
    eye8 = (lax.broadcasted_iota(jnp.int32, (8, 8), 0)
            == lax.broadcasted_iota(jnp.int32, (8, 8), 1))
    tr2 = jnp.sum(jnp.where(eye8, adjp2, 0.0))
    link2sq = jnp.sum(adjp * adjp) - 2.0 * tr2 + jnp.sum(sts2 * sts2)
    link2 = jnp.sqrt(jnp.maximum(link2sq, 1e-12)) / 32.0
    reg2 = link2 + ent2

    sts1 = sts_ref[...]
    eye32 = (lax.broadcasted_iota(jnp.int32, (32, 32), 0)
             == lax.broadcasted_iota(jnp.int32, (32, 32), 1))
    tr1 = jnp.sum(jnp.where(eye32, adjp, 0.0))
    link1sq = fro_ref[0, 0] - 2.0 * tr1 + jnp.sum(sts1 * sts1)
    link1 = jnp.sqrt(jnp.maximum(link1sq, 1e-12)) / 10000.0
    ent1 = ent_ref[0, 0] / 10000.0
    reg = (link1 + ent1) * 10.0 + reg2 * 0.1

    acc = bf1_ref[...]
    for r in range(8):
        acc = acc + jnp.dot(x3[r:r + 1, :], wf1_ref[r],
                            preferred_element_type=jnp.float32)
    h1f = jnp.maximum(acc, 0.0)
    out2 = jnp.dot(h1f, wf2_ref[...], preferred_element_type=jnp.float32) \
        + bf2_ref[...]
    outa_ref[...] = jnp.broadcast_to(out2, (8, 128))
    outb_ref[...] = jnp.full((8, 128), reg)


def _tail(adjp, xp, sts, fro, ent, wp2, asp2, adp2, aep2, wf1g, bf1p, wf2p,
          bf2p):
    vm = pl.BlockSpec(memory_space=pltpu.VMEM)
    sm = pl.BlockSpec(memory_space=pltpu.SMEM)
    return pl.pallas_call(
        _tail_body,
        in_specs=[vm, vm, vm, sm, sm, vm, vm, vm, sm, vm, vm, vm, vm],
        out_shape=[
            jax.ShapeDtypeStruct((8, 128), jnp.float32),
            jax.ShapeDtypeStruct((8, 128), jnp.float32),
        ],
    )(adjp, xp, sts, fro, ent, wp2, asp2, adp2, aep2, wf1g, bf1p, wf2p, bf2p)


NW = 32
EC = E_PAD // NW
NCH = EC // 128
NROWS = N_PAD // 16


def _sc_mesh():
    return plsc.VectorSubcoreMesh(core_axis_name="c", subcore_axis_name="s")


def _sc_phase_a(ss, sd, et, k32, srcp, dstp):
    @functools.partial(
        pl.kernel,
        out_type=[
            jax.ShapeDtypeStruct((2, N_PAD, 32), jnp.float32),
            jax.ShapeDtypeStruct((E_PAD, 32), jnp.float32),
        ],
        mesh=_sc_mesh(),
        compiler_params=pltpu.CompilerParams(use_tc_tiling_on_sc=False),
        scratch_types=[
            pltpu.VMEM((NCH, 128), jnp.int32),
            pltpu.VMEM((NCH, 128), jnp.int32),
            pltpu.VMEM((128, 32), jnp.float32),
            pltpu.VMEM((128, 32), jnp.float32),
            pltpu.VMEM((128, 32), jnp.float32),
            pltpu.VMEM((128, 32), jnp.float32),
            pltpu.VMEM((32,), jnp.float32),
            pltpu.VMEM((NROWS, 32), jnp.float32),
            pltpu.VMEM_SHARED((N_PAD, 32), jnp.float32),
            pltpu.SemaphoreType.DMA,
            pltpu.SemaphoreType.DMA,
        ],
    )
    def k(ss_hbm, sd_hbm, et_hbm, k_hbm, src_hbm, dst_hbm, den_out, ex_out,
          srci, dsti, ssb, sdb, etb, exb, kv, stage, den_sh, sem1, sem2):
        cid = lax.axis_index("c")
        sid = lax.axis_index("s")
        wid = sid * 2 + cid
        base = wid * EC

        pltpu.sync_copy(k_hbm, kv)
        k0 = kv[0:16]
        k1 = kv[16:32]

        def zbody(i, _):
            stage[i, 0:16] = jnp.zeros((16,), jnp.float32)
            stage[i, 16:32] = jnp.zeros((16,), jnp.float32)
            return 0

        lax.fori_loop(0, NROWS, zbody, 0)
        pltpu.sync_copy(stage, den_sh.at[pl.ds(sid * NROWS, NROWS)])
        plsc.subcore_barrier()

        def chunk(c, _):
            off = base + c * 128
            pltpu.sync_copy(src_hbm.at[pl.ds(off, 128)], srci.at[c])
            pltpu.sync_copy(dst_hbm.at[pl.ds(off, 128)], dsti.at[c])
            pltpu.async_copy(ss_hbm.at[srci.at[c]], ssb, sem1).wait()
            pltpu.async_copy(sd_hbm.at[dsti.at[c]], sdb, sem2).wait()
            pltpu.sync_copy(et_hbm.at[pl.ds(off, 128), :], etb)

            def jbody(j, _):
                a0 = ssb[j, 0:16] + sdb[j, 0:16] + etb[j, 0:16]
                a0 = jnp.maximum(a0, 0.2 * a0) - k0
                exb[j, 0:16] = jnp.exp(a0)
                a1 = ssb[j, 16:32] + sdb[j, 16:32] + etb[j, 16:32]
                a1 = jnp.maximum(a1, 0.2 * a1) - k1
                exb[j, 16:32] = jnp.exp(a1)
                return 0

            lax.fori_loop(0, 128, jbody, 0)
            pltpu.sync_copy(exb, den_sh.at[dsti.at[c]], add=True)
            pltpu.sync_copy(exb, ex_out.at[pl.ds(off, 128), :])
            return 0

        lax.fori_loop(0, NCH, chunk, 0)
        plsc.subcore_barrier()
        pltpu.sync_copy(
            den_sh.at[pl.ds(sid * NROWS, NROWS)],
            den_out.at[cid, pl.ds(sid * NROWS, NROWS), :])

    return k(ss, sd, et, k32, srcp, dstp)


def _sc_phase_b(ex, rden, h, srcp, dstp):
    @functools.partial(
        pl.kernel,
        out_type=jax.ShapeDtypeStruct((2, N_PAD, 32), jnp.float32),
        mesh=_sc_mesh(),
        compiler_params=pltpu.CompilerParams(use_tc_tiling_on_sc=False),
        scratch_types=[
            pltpu.VMEM((NCH, 128), jnp.int32),
            pltpu.VMEM((NCH, 128), jnp.int32),
            pltpu.VMEM((128, 32), jnp.float32),
            pltpu.VMEM((128, 32), jnp.float32),
            pltpu.VMEM((128, 32), jnp.float32),
            pltpu.VMEM((128, 32), jnp.float32),
            pltpu.VMEM((NROWS, 32), jnp.float32),
            pltpu.VMEM_SHARED((N_PAD, 32), jnp.float32),
            pltpu.SemaphoreType.DMA,
            pltpu.SemaphoreType.DMA,
        ],
    )
    def k(ex_hbm, rd_hbm, h_hbm, src_hbm, dst_hbm, agg_out,
          srci, dsti, exb, rdb, hb, msgb, stage, agg_sh, sem1, sem2):
        cid = lax.axis_index("c")
        sid = lax.axis_index("s")
        wid = sid * 2 + cid
        base = wid * EC

        def zbody(i, _):
            stage[i, 0:16] = jnp.zeros((16,), jnp.float32)
            stage[i, 16:32] = jnp.zeros((16,), jnp.float32)
            return 0

        lax.fori_loop(0, NROWS, zbody, 0)
        pltpu.sync_copy(stage, agg_sh.at[pl.ds(sid * NROWS, NROWS)])
        plsc.subcore_barrier()

        def chunk(c, _):
            off = base + c * 128
            pltpu.sync_copy(src_hbm.at[pl.ds(off, 128)], srci.at[c])
            pltpu.sync_copy(dst_hbm.at[pl.ds(off, 128)], dsti.at[c])
            pltpu.async_copy(rd_hbm.at[dsti.at[c]], rdb, sem1).wait()
            pltpu.async_copy(h_hbm.at[srci.at[c]], hb, sem2).wait()
            pltpu.sync_copy(ex_hbm.at[pl.ds(off, 128), :], exb)

            def jbody(j, _):
                msgb[j, 0:16] = hb[j, 0:16] * exb[j, 0:16] * rdb[j, 0:16]
                msgb[j, 16:32] = hb[j, 16:32] * exb[j, 16:32] * rdb[j, 16:32]
                return 0

            lax.fori_loop(0, 128, jbody, 0)
            pltpu.sync_copy(msgb, agg_sh.at[dsti.at[c]], add=True)
            return 0

        lax.fori_loop(0, NCH, chunk, 0)
        plsc.subcore_barrier()
        pltpu.sync_copy(
            agg_sh.at[pl.ds(sid * NROWS, NROWS)],
            agg_out.at[cid, pl.ds(sid * NROWS, NROWS), :])

    return k(ex, rden, h, srcp, dstp)


def _conv_segops(ss, sd, et, k32, h, srcp, dstp):
    dparts, ex = _sc_phase_a(ss, sd, et, k32, srcp, dstp)
    rden = _rden(dparts)
    return _sc_phase_b(ex, rden, h, srcp, dstp)


def kernel(x, edge_index, edge_attr, y, adj, W1, a_src1, a_dst1, a_edge1,
           Wp1, a_src_p1, a_dst_p1, a_edge_p1, Wp2, a_src_p2, a_dst_p2,
           a_edge_p2, Wf1, bf1, Wf2, bf2):
    n, dfeat = x.shape
    e = edge_index.shape[1]

    xp_in = jnp.pad(x, ((0, N_PAD - n), (0, 0)))
    srcp = jnp.concatenate(
        [edge_index[0].astype(jnp.int32),
         jnp.zeros((E_PAD - e,), jnp.int32)])
    dstp = jnp.concatenate(
        [edge_index[1].astype(jnp.int32),
         jnp.full((E_PAD - e,), n, jnp.int32)])
    eap = jnp.pad(edge_attr, ((0, E_PAD - e), (0, 0)))

    w1p = jnp.pad(W1, ((0, 0), (0, 2)))
    asrc1 = jnp.zeros((32, 32), jnp.float32)
    adst1 = jnp.zeros((32, 32), jnp.float32)
    ae1p = jnp.zeros((4, 32), jnp.float32)
    for hh in range(5):
        blk_s = jnp.broadcast_to(a_src1[hh][:, None], (6, 6))
        blk_d = jnp.broadcast_to(a_dst1[hh][:, None], (6, 6))
        asrc1 = asrc1.at[hh * 6:(hh + 1) * 6, hh * 6:(hh + 1) * 6].set(blk_s)
        adst1 = adst1.at[hh * 6:(hh + 1) * 6, hh * 6:(hh + 1) * 6].set(blk_d)
        ae1p = ae1p.at[:, hh * 6:(hh + 1) * 6].set(
            jnp.broadcast_to(a_edge1[:, hh:hh + 1], (4, 6)))

    wp1p = jnp.pad(Wp1, ((0, 2), (0, 0)))
    asrc2 = jnp.broadcast_to(a_src_p1[0][:, None], (32, 32))
    adst2 = jnp.broadcast_to(a_dst_p1[0][:, None], (32, 32))
    ae2p = jnp.broadcast_to(a_edge_p1, (4, 32))

    h1p, ss1, sd1 = _node_prep(xp_in, w1p, asrc1, adst1)
    et1, et2 = _edge_prep(eap, ae1p, ae2p)
    k1 = _maxes(ss1, sd1, et1)[0]

    aggparts1 = _conv_segops(ss1, sd1, et1, k1, h1p, srcp, dstp)

    x1, h2p, ss2, sd2 = _prep2(aggparts1, wp1p, asrc2, adst2)
    k2 = _maxes(ss2, sd2, et2)[0]

    aggparts2 = _conv_segops(ss2, sd2, et2, k2, h2p, srcp, dstp)

    s, ent_sum = _smax(aggparts2, n)
    adjp, xp, sts, fro = _adj_pass(adj, s, x1)

    wp2p = jnp.pad(Wp2, ((0, 2), (0, 4)))
    asp2 = jnp.zeros((8, 8), jnp.float32).at[0:4, 0].set(a_src_p2[0])
    adp2 = jnp.zeros((8, 8), jnp.float32).at[0:4, 0].set(a_dst_p2[0])
    aep2 = a_edge_p2.reshape(1, 1)
    wf1g = jnp.zeros((8, 32, 32), jnp.float32).at[0:4, 0:30, :].set(
        Wf1.reshape(4, 30, 32))
    bf1p = bf1.reshape(1, 32)
    wf2p = jnp.zeros((32, 128), jnp.float32).at[:, 0:2].set(Wf2)
    bf2p = jnp.zeros((1, 128), jnp.float32).at[0, 0:2].set(bf2)

    outa, outb = _tail(adjp, xp, sts, fro, ent_sum, wp2p, asp2, adp2, aep2,
                       wf1g, bf1p, wf2p, bf2p)
    return outa[0:1, 0:2], outb[0, 0]


# speedup vs baseline: 14.5751x; 1.0769x over previous
import functools

import jax
import jax.numpy as jnp
from jax import lax
from jax.experimental import pallas as pl
from jax.experimental.pallas import tpu as pltpu
from jax.experimental.pallas import tpu_sc as plsc

N_PAD = 10240
E_PAD = 163840
BN = 1024
BE = 2048
ADJ_BR = 400
ADJ_BC = 1024


def _node_prep_body(x_ref, w_ref, asrc_ref, adst_ref, h_ref, ss_ref, sd_ref):
    h = jnp.dot(x_ref[...], w_ref[...], preferred_element_type=jnp.float32)
    h_ref[...] = h
    ss_ref[...] = jnp.dot(h, asrc_ref[...], preferred_element_type=jnp.float32)
    sd_ref[...] = jnp.dot(h, adst_ref[...], preferred_element_type=jnp.float32)


def _node_prep(xp, w, asrc, adst):
    npad, din = xp.shape
    dh = w.shape[1]
    grid = npad // BN
    return pl.pallas_call(
        _node_prep_body,
        grid=(grid,),
        in_specs=[
            pl.BlockSpec((BN, din), lambda i: (i, 0)),
            pl.BlockSpec((din, dh), lambda i: (0, 0)),
            pl.BlockSpec((dh, 32), lambda i: (0, 0)),
            pl.BlockSpec((dh, 32), lambda i: (0, 0)),
        ],
        out_specs=[
            pl.BlockSpec((BN, dh), lambda i: (i, 0)),
            pl.BlockSpec((BN, 32), lambda i: (i, 0)),
            pl.BlockSpec((BN, 32), lambda i: (i, 0)),
        ],
        out_shape=[
            jax.ShapeDtypeStruct((npad, dh), jnp.float32),
            jax.ShapeDtypeStruct((npad, 32), jnp.float32),
            jax.ShapeDtypeStruct((npad, 32), jnp.float32),
        ],
    )(xp, w, asrc, adst)


def _edge_prep_body(ea_ref, ae1_ref, ae2_ref, et1_ref, et2_ref):
    ea = ea_ref[...]
    et1_ref[...] = jnp.dot(ea, ae1_ref[...], preferred_element_type=jnp.float32)
    et2_ref[...] = jnp.dot(ea, ae2_ref[...], preferred_element_type=jnp.float32)


def _edge_prep(eap, ae1, ae2):
    epad, de = eap.shape
    grid = epad // BE
    return pl.pallas_call(
        _edge_prep_body,
        grid=(grid,),
        in_specs=[
            pl.BlockSpec((BE, de), lambda i: (i, 0)),
            pl.BlockSpec((de, 32), lambda i: (0, 0)),
            pl.BlockSpec((de, 32), lambda i: (0, 0)),
        ],
        out_specs=[
            pl.BlockSpec((BE, 32), lambda i: (i, 0)),
            pl.BlockSpec((BE, 32), lambda i: (i, 0)),
        ],
        out_shape=[
            jax.ShapeDtypeStruct((epad, 32), jnp.float32),
            jax.ShapeDtypeStruct((epad, 32), jnp.float32),
        ],
    )(eap, ae1, ae2)


def _maxes_body(ss_ref, sd_ref, et_ref, k_ref, a1_ref, a2_ref, a3_ref, *, ng):
    i = pl.program_id(0)

    @pl.when(i == 0)
    def _():
        a1_ref[...] = jnp.full_like(a1_ref, -1e30)
        a2_ref[...] = jnp.full_like(a2_ref, -1e30)
        a3_ref[...] = jnp.full_like(a3_ref, -1e30)

    def colmax(r):
        return jnp.broadcast_to(jnp.max(r[...], axis=0)[None, :], (8, 32))

    a1_ref[...] = jnp.maximum(a1_ref[...], colmax(ss_ref))
    a2_ref[...] = jnp.maximum(a2_ref[...], colmax(sd_ref))
    a3_ref[...] = jnp.maximum(a3_ref[...], colmax(et_ref))

    @pl.when(i == ng - 1)
    def _():
        m = a1_ref[...] + a2_ref[...] + a3_ref[...]
        k_ref[...] = jnp.maximum(m, 0.2 * m)


def _maxes(ss, sd, et):
    npad = ss.shape[0]
    epad = et.shape[0]
    nb = npad // BN
    ng = epad // BE
    return pl.pallas_call(
        functools.partial(_maxes_body, ng=ng),
        grid=(ng,),
        in_specs=[
            pl.BlockSpec((BN, 32), lambda i: (i % nb, 0)),
            pl.BlockSpec((BN, 32), lambda i: (i % nb, 0)),
            pl.BlockSpec((BE, 32), lambda i: (i, 0)),
        ],
        out_specs=pl.BlockSpec((8, 32), lambda i: (0, 0)),
        out_shape=jax.ShapeDtypeStruct((8, 32), jnp.float32),
        scratch_shapes=[
            pltpu.VMEM((8, 32), jnp.float32),
            pltpu.VMEM((8, 32), jnp.float32),
            pltpu.VMEM((8, 32), jnp.float32),
        ],
    )(ss, sd, et)


def _rden_body(d_ref, r_ref):
    d = d_ref[0] + d_ref[1]
    r_ref[...] = 1.0 / (d + 1e-16)


def _rden(dparts):
    npad = dparts.shape[1]
    return pl.pallas_call(
        _rden_body,
        out_shape=jax.ShapeDtypeStruct((npad, 32), jnp.float32),
    )(dparts)


def _prep2_body(a_ref, w_ref, asrc_ref, adst_ref, x1_ref, h2_ref, ss_ref,
                sd_ref):
    x1 = a_ref[0] + a_ref[1]
    x1_ref[...] = x1
    h2 = jnp.dot(x1, w_ref[...], preferred_element_type=jnp.float32)
    h2_ref[...] = h2
    ss_ref[...] = jnp.dot(h2, asrc_ref[...], preferred_element_type=jnp.float32)
    sd_ref[...] = jnp.dot(h2, adst_ref[...], preferred_element_type=jnp.float32)


def _prep2(aggparts, wp, asrc, adst):
    npad = aggparts.shape[1]
    grid = npad // BN
    return pl.pallas_call(
        _prep2_body,
        grid=(grid,),
        in_specs=[
            pl.BlockSpec((2, BN, 32), lambda i: (0, i, 0)),
            pl.BlockSpec((32, 32), lambda i: (0, 0)),
            pl.BlockSpec((32, 32), lambda i: (0, 0)),
            pl.BlockSpec((32, 32), lambda i: (0, 0)),
        ],
        out_specs=[
            pl.BlockSpec((BN, 32), lambda i: (i, 0)),
            pl.BlockSpec((BN, 32), lambda i: (i, 0)),
            pl.BlockSpec((BN, 32), lambda i: (i, 0)),
            pl.BlockSpec((BN, 32), lambda i: (i, 0)),
        ],
        out_shape=[
            jax.ShapeDtypeStruct((npad, 32), jnp.float32),
            jax.ShapeDtypeStruct((npad, 32), jnp.float32),
            jax.ShapeDtypeStruct((npad, 32), jnp.float32),
            jax.ShapeDtypeStruct((npad, 32), jnp.float32),
        ],
    )(aggparts, wp, asrc, adst)


def _smax_body(s1_ref, s_ref, ent_ref, *, nreal):
    i = pl.program_id(0)
    z = s1_ref[0] + s1_ref[1]
    m = jnp.max(z, axis=1, keepdims=True)
    e = jnp.exp(z - m)
    sm = e / jnp.sum(e, axis=1, keepdims=True)
    rid = i * BN + lax.broadcasted_iota(jnp.int32, sm.shape, 0)
    sm = jnp.where(rid < nreal, sm, 0.0)
    s_ref[...] = sm
    ent = -jnp.sum(sm * jnp.log(sm + 1e-15))

    @pl.when(i == 0)
    def _():
        ent_ref[0, 0] = 0.0

    ent_ref[0, 0] += ent


def _smax(s1parts, nreal):
    npad = s1parts.shape[1]
    grid = npad // BN
    return pl.pallas_call(
        functools.partial(_smax_body, nreal=nreal),
        grid=(grid,),
        in_specs=[pl.BlockSpec((2, BN, 32), lambda i: (0, i, 0))],
        out_specs=[
            pl.BlockSpec((BN, 32), lambda i: (i, 0)),
            pl.BlockSpec((1, 1), lambda i: (0, 0),
                         memory_space=pltpu.SMEM),
        ],
        out_shape=[
            jax.ShapeDtypeStruct((npad, 32), jnp.float32),
            jax.ShapeDtypeStruct((1, 1), jnp.float32),
        ],
    )(s1parts)


def _adj_body(adj_ref, sk_ref, si_ref, x1_ref, adjp_ref, xp_ref, sts_ref,
              fro_ref, tmp_ref, *, nk, ncols):
    i = pl.program_id(0)
    k = pl.program_id(1)

    blk = adj_ref[...]
    colid = k * ADJ_BC + lax.broadcasted_iota(jnp.int32, blk.shape, 1)
    blk = jnp.where(colid < ncols, blk, 0.0)

    @pl.when(jnp.logical_and(i == 0, k == 0))
    def _():
        adjp_ref[...] = jnp.zeros_like(adjp_ref)
        xp_ref[...] = jnp.zeros_like(xp_ref)
        sts_ref[...] = jnp.zeros_like(sts_ref)
        fro_ref[0, 0] = 0.0

    fro_ref[0, 0] += jnp.sum(blk * blk)

    part = jnp.dot(blk, sk_ref[...], preferred_element_type=jnp.float32)

    @pl.when(k == 0)
    def _():
        tmp_ref[...] = part
        si = si_ref[...]
        xp_ref[...] += lax.dot_general(
            si, x1_ref[...], (((0,), (0,)), ((), ())),
            preferred_element_type=jnp.float32)
        sts_ref[...] += lax.dot_general(
            si, si, (((0,), (0,)), ((), ())),
            preferred_element_type=jnp.float32)

    @pl.when(k > 0)
    def _():
        tmp_ref[...] += part

    @pl.when(k == nk - 1)
    def _():
        adjp_ref[...] += lax.dot_general(
            si_ref[...], tmp_ref[...], (((0,), (0,)), ((), ())),
            preferred_element_type=jnp.float32)


def _adj_pass(adj, s, x1):
    nrows, ncols = adj.shape
    ni = nrows // ADJ_BR
    nk = (ncols + ADJ_BC - 1) // ADJ_BC
    return pl.pallas_call(
        functools.partial(_adj_body, nk=nk, ncols=ncols),
        grid=(ni, nk),
        in_specs=[
            pl.BlockSpec((ADJ_BR, ADJ_BC), lambda i, k: (i, k)),
            pl.BlockSpec((ADJ_BC, 32), lambda i, k: (k, 0)),
            pl.BlockSpec((ADJ_BR, 32), lambda i, k: (i, 0)),
            pl.BlockSpec((ADJ_BR, 32), lambda i, k: (i, 0)),
        ],
        out_specs=[
            pl.BlockSpec((32, 32), lambda i, k: (0, 0)),
            pl.BlockSpec((32, 32), lambda i, k: (0, 0)),
            pl.BlockSpec((32, 32), lambda i, k: (0, 0)),
            pl.BlockSpec((1, 1), lambda i, k: (0, 0),
                         memory_space=pltpu.SMEM),
        ],
        out_shape=[
            jax.ShapeDtypeStruct((32, 32), jnp.float32),
            jax.ShapeDtypeStruct((32, 32), jnp.float32),
            jax.ShapeDtypeStruct((32, 32), jnp.float32),
            jax.ShapeDtypeStruct((1, 1), jnp.float32),
        ],
        scratch_shapes=[pltpu.VMEM((ADJ_BR, 32), jnp.float32)],
        compiler_params=pltpu.CompilerParams(
            dimension_semantics=("arbitrary", "arbitrary")),
    )(adj, s, s, x1)


def _tail_body(adjp_ref, xp_ref, sts_ref, fro_ref, ent_ref, wp2_ref, asp2_ref,
               adp2_ref, aep2_ref, wf1_ref, bf1_ref, wf2_ref, bf2_ref,
               outa_ref, outb_ref):
    adjp = adjp_ref[...]
    x2 = xp_ref[...]
    h3 = jnp.dot(x2, wp2_ref[...], preferred_element_type=jnp.float32)
    ss3 = jnp.dot(h3, asp2_ref[...], preferred_element_type=jnp.float32)
    sd3m = lax.dot_general(adp2_ref[...], h3, (((0,), (1,)), ((), ())),
                           preferred_element_type=jnp.float32)
    alpha = ss3[:, 0:1] + sd3m[0:1, :] + adjp * aep2_ref[0, 0]
    alpha = jnp.maximum(alpha, 0.2 * alpha)
    cmax = jnp.max(alpha, axis=0, keepdims=True)
    ex = jnp.exp(alpha - cmax)
    att = ex / (jnp.sum(ex, axis=0, keepdims=True) + 1e-16)
    s2 = lax.dot_general(att, h3, (((0,), (0,)), ((), ())),
                         preferred_element_type=jnp.float32)

    colmask = lax.broadcasted_iota(jnp.int32, s2.shape, 1) < 4
    z = jnp.where(colmask, s2, -1e30)
    m2 = jnp.max(z, axis=1, keepdims=True)
    e2 = jnp.where(colmask, jnp.exp(z - m2), 0.0)
    s2s = e2 / jnp.sum(e2, axis=1, keepdims=True)
    ent2 = -jnp.sum(s2s * jnp.log(s2s + 1e-15)) / 32.0

    x3 = lax.dot_general(s2s, x2, (((0,), (0,)), ((), ())),
                         preferred_element_type=jnp.float32)
    adjs2 = jnp.dot(adjp, s2s, preferred_element_type=jnp.float32)
    adjp2 = lax.dot_general(s2s, adjs2, (((0,), (0,)), ((), ())),
                            preferred_element_type=jnp.float32)
    sts2 = lax.dot_general(s2s, s2s, (((0,), (0,)), ((), ())),
                           preferred_element_type=jnp.float32)
    eye8 = (lax.broadcasted_iota(jnp.int32, (8, 8), 0)
            == lax.broadcasted_iota(jnp.int32, (8, 8), 1))
    tr2 = jnp.sum(jnp.where(eye8, adjp2, 0.0))
    link2sq = jnp.sum(adjp * adjp) - 2.0 * tr2 + jnp.sum(sts2 * sts2)
    link2 = jnp.sqrt(jnp.maximum(link2sq, 1e-12)) / 32.0
    reg2 = link2 + ent2

    sts1 = sts_ref[...]
    eye32 = (lax.broadcasted_iota(jnp.int32, (32, 32), 0)
             == lax.broadcasted_iota(jnp.int32, (32, 32), 1))
    tr1 = jnp.sum(jnp.where(eye32, adjp, 0.0))
    link1sq = fro_ref[0, 0] - 2.0 * tr1 + jnp.sum(sts1 * sts1)
    link1 = jnp.sqrt(jnp.maximum(link1sq, 1e-12)) / 10000.0
    ent1 = ent_ref[0, 0] / 10000.0
    reg = (link1 + ent1) * 10.0 + reg2 * 0.1

    acc = bf1_ref[...]
    for r in range(8):
        acc = acc + jnp.dot(x3[r:r + 1, :], wf1_ref[r],
                            preferred_element_type=jnp.float32)
    h1f = jnp.maximum(acc, 0.0)
    out2 = jnp.dot(h1f, wf2_ref[...], preferred_element_type=jnp.float32) \
        + bf2_ref[...]
    outa_ref[...] = jnp.broadcast_to(out2, (8, 128))
    outb_ref[...] = jnp.full((8, 128), reg)


def _tail(adjp, xp, sts, fro, ent, wp2, asp2, adp2, aep2, wf1g, bf1p, wf2p,
          bf2p):
    vm = pl.BlockSpec(memory_space=pltpu.VMEM)
    sm = pl.BlockSpec(memory_space=pltpu.SMEM)
    return pl.pallas_call(
        _tail_body,
        in_specs=[vm, vm, vm, sm, sm, vm, vm, vm, sm, vm, vm, vm, vm],
        out_shape=[
            jax.ShapeDtypeStruct((8, 128), jnp.float32),
            jax.ShapeDtypeStruct((8, 128), jnp.float32),
        ],
    )(adjp, xp, sts, fro, ent, wp2, asp2, adp2, aep2, wf1g, bf1p, wf2p, bf2p)


NW = 32
EC = E_PAD // NW
NCH = EC // 128
NROWS = N_PAD // 16


def _sc_mesh():
    return plsc.VectorSubcoreMesh(core_axis_name="c", subcore_axis_name="s")


def _sc_phase_a(ss, sd, et, k32, src2d, dst2d):
    @functools.partial(
        pl.kernel,
        out_type=[
            jax.ShapeDtypeStruct((2, N_PAD, 32), jnp.float32),
            jax.ShapeDtypeStruct((E_PAD, 32), jnp.float32),
        ],
        mesh=_sc_mesh(),
        compiler_params=pltpu.CompilerParams(use_tc_tiling_on_sc=False),
        scratch_types=[
            pltpu.VMEM((NCH, 128), jnp.int32),
            pltpu.VMEM((NCH, 128), jnp.int32),
            pltpu.VMEM((128, 32), jnp.float32),
            pltpu.VMEM((128, 32), jnp.float32),
            pltpu.VMEM((128, 32), jnp.float32),
            pltpu.VMEM((128, 32), jnp.float32),
            pltpu.VMEM((128, 32), jnp.float32),
            pltpu.VMEM((128, 32), jnp.float32),
            pltpu.VMEM((128, 32), jnp.float32),
            pltpu.VMEM((128, 32), jnp.float32),
            pltpu.VMEM((32,), jnp.float32),
            pltpu.VMEM((NROWS, 32), jnp.float32),
            pltpu.VMEM_SHARED((N_PAD, 32), jnp.float32),
            pltpu.SemaphoreType.DMA,
            pltpu.SemaphoreType.DMA,
            pltpu.SemaphoreType.DMA,
            pltpu.SemaphoreType.DMA,
            pltpu.SemaphoreType.DMA,
            pltpu.SemaphoreType.DMA,
            pltpu.SemaphoreType.DMA,
            pltpu.SemaphoreType.DMA,
        ],
    )
    def k(ss_hbm, sd_hbm, et_hbm, k_hbm, src_hbm, dst_hbm, den_out, ex_out,
          srci, dsti, ssb0, sdb0, etb0, exb0, ssb1, sdb1, etb1, exb1,
          kv, stage, den_sh, gsem0, gsem1, lsem0, lsem1, ssem0, ssem1,
          wsem0, wsem1):
        cid = lax.axis_index("c")
        sid = lax.axis_index("s")
        wid = sid * 2 + cid
        base = wid * EC

        pltpu.sync_copy(k_hbm, kv)
        pltpu.sync_copy(src_hbm.at[pl.ds(wid * NCH, NCH), :], srci)
        pltpu.sync_copy(dst_hbm.at[pl.ds(wid * NCH, NCH), :], dsti)
        k0 = kv[0:16]
        k1 = kv[16:32]

        def zbody(i, _):
            stage[i, 0:16] = jnp.zeros((16,), jnp.float32)
            stage[i, 16:32] = jnp.zeros((16,), jnp.float32)
            return 0

        lax.fori_loop(0, NROWS, zbody, 0)
        pltpu.sync_copy(stage, den_sh.at[pl.ds(sid * NROWS, NROWS)])
        plsc.subcore_barrier()

        def issue_in(r, ssb, sdb, etb, gsem, lsem):
            off = base + r * 128
            d1 = pltpu.async_copy(ss_hbm.at[srci.at[r]], ssb, gsem)
            d2 = pltpu.async_copy(sd_hbm.at[dsti.at[r]], sdb, gsem)
            d3 = pltpu.async_copy(et_hbm.at[pl.ds(off, 128), :], etb, lsem)
            return d1, d2, d3

        def compute(ssb, sdb, etb, exb):
            def jbody(j, _):
                a0 = ssb[j, 0:16] + sdb[j, 0:16] + etb[j, 0:16]
                a0 = jnp.maximum(a0, 0.2 * a0) - k0
                exb[j, 0:16] = jnp.exp(a0)
                a1 = ssb[j, 16:32] + sdb[j, 16:32] + etb[j, 16:32]
                a1 = jnp.maximum(a1, 0.2 * a1) - k1
                exb[j, 16:32] = jnp.exp(a1)
                return 0

            lax.fori_loop(0, 128, jbody, 0, unroll=4)

        def issue_out(r, exb, ssem, wsem):
            off = base + r * 128
            o1 = pltpu.async_copy(exb, den_sh.at[dsti.at[r]], ssem, add=True)
            o2 = pltpu.async_copy(exb, ex_out.at[pl.ds(off, 128), :], wsem)
            return o1, o2

        def chunk2(c, _):
            ra = 2 * c
            rb = 2 * c + 1
            ia = issue_in(ra, ssb0, sdb0, etb0, gsem0, lsem0)
            ib = issue_in(rb, ssb1, sdb1, etb1, gsem1, lsem1)
            for d in ia:
                d.wait()
            compute(ssb0, sdb0, etb0, exb0)
            oa = issue_out(ra, exb0, ssem0, wsem0)
            for d in ib:
                d.wait()
            compute(ssb1, sdb1, etb1, exb1)
            ob = issue_out(rb, exb1, ssem1, wsem1)
            for d in oa:
                d.wait()
            for d in ob:
                d.wait()
            return 0

        lax.fori_loop(0, NCH // 2, chunk2, 0)
        plsc.subcore_barrier()
        pltpu.sync_copy(
            den_sh.at[pl.ds(sid * NROWS, NROWS)],
            den_out.at[cid, pl.ds(sid * NROWS, NROWS), :])

    return k(ss, sd, et, k32, src2d, dst2d)


def _sc_phase_b(ex, rden, h, src2d, dst2d):
    @functools.partial(
        pl.kernel,
        out_type=jax.ShapeDtypeStruct((2, N_PAD, 32), jnp.float32),
        mesh=_sc_mesh(),
        compiler_params=pltpu.CompilerParams(use_tc_tiling_on_sc=False),
        scratch_types=[
            pltpu.VMEM((NCH, 128), jnp.int32),
            pltpu.VMEM((NCH, 128), jnp.int32),
            pltpu.VMEM((128, 32), jnp.float32),
            pltpu.VMEM((128, 32), jnp.float32),
            pltpu.VMEM((128, 32), jnp.float32),
            pltpu.VMEM((128, 32), jnp.float32),
            pltpu.VMEM((128, 32), jnp.float32),
            pltpu.VMEM((128, 32), jnp.float32),
            pltpu.VMEM((128, 32), jnp.float32),
            pltpu.VMEM((128, 32), jnp.float32),
            pltpu.VMEM((NROWS, 32), jnp.float32),
            pltpu.VMEM_SHARED((N_PAD, 32), jnp.float32),
            pltpu.SemaphoreType.DMA,
            pltpu.SemaphoreType.DMA,
            pltpu.SemaphoreType.DMA,
            pltpu.SemaphoreType.DMA,
            pltpu.SemaphoreType.DMA,
            pltpu.SemaphoreType.DMA,
        ],
    )
    def k(ex_hbm, rd_hbm, h_hbm, src_hbm, dst_hbm, agg_out,
          srci, dsti, exb0, rdb0, hb0, msgb0, exb1, rdb1, hb1, msgb1,
          stage, agg_sh, gsem0, gsem1, lsem0, lsem1, ssem0, ssem1):
        cid = lax.axis_index("c")
        sid = lax.axis_index("s")
        wid = sid * 2 + cid
        base = wid * EC

        pltpu.sync_copy(src_hbm.at[pl.ds(wid * NCH, NCH), :], srci)
        pltpu.sync_copy(dst_hbm.at[pl.ds(wid * NCH, NCH), :], dsti)

        def zbody(i, _):
            stage[i, 0:16] = jnp.zeros((16,), jnp.float32)
            stage[i, 16:32] = jnp.zeros((16,), jnp.float32)
            return 0

        lax.fori_loop(0, NROWS, zbody, 0)
        pltpu.sync_copy(stage, agg_sh.at[pl.ds(sid * NROWS, NROWS)])
        plsc.subcore_barrier()

        def issue_in(r, rdb, hb, exb, gsem, lsem):
            off = base + r * 128
            d1 = pltpu.async_copy(rd_hbm.at[dsti.at[r]], rdb, gsem)
            d2 = pltpu.async_copy(h_hbm.at[srci.at[r]], hb, gsem)
            d3 = pltpu.async_copy(ex_hbm.at[pl.ds(off, 128), :], exb, lsem)
            return d1, d2, d3

        def compute(rdb, hb, exb, msgb):
            def jbody(j, _):
                msgb[j, 0:16] = hb[j, 0:16] * exb[j, 0:16] * rdb[j, 0:16]
                msgb[j, 16:32] = hb[j, 16:32] * exb[j, 16:32] * rdb[j, 16:32]
                return 0

            lax.fori_loop(0, 128, jbody, 0, unroll=4)

        def chunk2(c, _):
            ra = 2 * c
            rb = 2 * c + 1
            ia = issue_in(ra, rdb0, hb0, exb0, gsem0, lsem0)
            ib = issue_in(rb, rdb1, hb1, exb1, gsem1, lsem1)
            for d in ia:
                d.wait()
            compute(rdb0, hb0, exb0, msgb0)
            oa = pltpu.async_copy(msgb0, agg_sh.at[dsti.at[ra]], ssem0,
                                  add=True)
            for d in ib:
                d.wait()
            compute(rdb1, hb1, exb1, msgb1)
            ob = pltpu.async_copy(msgb1, agg_sh.at[dsti.at[rb]], ssem1,
                                  add=True)
            oa.wait()
            ob.wait()
            return 0

        lax.fori_loop(0, NCH // 2, chunk2, 0)
        plsc.subcore_barrier()
        pltpu.sync_copy(
            agg_sh.at[pl.ds(sid * NROWS, NROWS)],
            agg_out.at[cid, pl.ds(sid * NROWS, NROWS), :])

    return k(ex, rden, h, src2d, dst2d)


def _conv_segops(ss, sd, et, k32, h, srcp, dstp):
    dparts, ex = _sc_phase_a(ss, sd, et, k32, srcp, dstp)
    rden = _rden(dparts)
    return _sc_phase_b(ex, rden, h, srcp, dstp)


def kernel(x, edge_index, edge_attr, y, adj, W1, a_src1, a_dst1, a_edge1,
           Wp1, a_src_p1, a_dst_p1, a_edge_p1, Wp2, a_src_p2, a_dst_p2,
           a_edge_p2, Wf1, bf1, Wf2, bf2):
    n, dfeat = x.shape
    e = edge_index.shape[1]

    xp_in = jnp.pad(x, ((0, N_PAD - n), (0, 0)))
    srcp = jnp.concatenate(
        [edge_index[0].astype(jnp.int32),
         jnp.zeros((E_PAD - e,), jnp.int32)]).reshape(E_PAD // 128, 128)
    dstp = jnp.concatenate(
        [edge_index[1].astype(jnp.int32),
         jnp.full((E_PAD - e,), n, jnp.int32)]).reshape(E_PAD // 128, 128)
    eap = jnp.pad(edge_attr, ((0, E_PAD - e), (0, 0)))

    w1p = jnp.pad(W1, ((0, 0), (0, 2)))
    asrc1 = jnp.zeros((32, 32), jnp.float32)
    adst1 = jnp.zeros((32, 32), jnp.float32)
    ae1p = jnp.zeros((4, 32), jnp.float32)
    for hh in range(5):
        blk_s = jnp.broadcast_to(a_src1[hh][:, None], (6, 6))
        blk_d = jnp.broadcast_to(a_dst1[hh][:, None], (6, 6))
        asrc1 = asrc1.at[hh * 6:(hh + 1) * 6, hh * 6:(hh + 1) * 6].set(blk_s)
        adst1 = adst1.at[hh * 6:(hh + 1) * 6, hh * 6:(hh + 1) * 6].set(blk_d)
        ae1p = ae1p.at[:, hh * 6:(hh + 1) * 6].set(
            jnp.broadcast_to(a_edge1[:, hh:hh + 1], (4, 6)))

    wp1p = jnp.pad(Wp1, ((0, 2), (0, 0)))
    asrc2 = jnp.broadcast_to(a_src_p1[0][:, None], (32, 32))
    adst2 = jnp.broadcast_to(a_dst_p1[0][:, None], (32, 32))
    ae2p = jnp.broadcast_to(a_edge_p1, (4, 32))

    h1p, ss1, sd1 = _node_prep(xp_in, w1p, asrc1, adst1)
    et1, et2 = _edge_prep(eap, ae1p, ae2p)
    k1 = _maxes(ss1, sd1, et1)[0]

    aggparts1 = _conv_segops(ss1, sd1, et1, k1, h1p, srcp, dstp)

    x1, h2p, ss2, sd2 = _prep2(aggparts1, wp1p, asrc2, adst2)
    k2 = _maxes(ss2, sd2, et2)[0]

    aggparts2 = _conv_segops(ss2, sd2, et2, k2, h2p, srcp, dstp)

    s, ent_sum = _smax(aggparts2, n)
    adjp, xp, sts, fro = _adj_pass(adj, s, x1)

    wp2p = jnp.pad(Wp2, ((0, 2), (0, 4)))
    asp2 = jnp.zeros((8, 8), jnp.float32).at[0:4, 0].set(a_src_p2[0])
    adp2 = jnp.zeros((8, 8), jnp.float32).at[0:4, 0].set(a_dst_p2[0])
    aep2 = a_edge_p2.reshape(1, 1)
    wf1g = jnp.zeros((8, 32, 32), jnp.float32).at[0:4, 0:30, :].set(
        Wf1.reshape(4, 30, 32))
    bf1p = bf1.reshape(1, 32)
    wf2p = jnp.zeros((32, 128), jnp.float32).at[:, 0:2].set(Wf2)
    bf2p = jnp.zeros((1, 128), jnp.float32).at[0, 0:2].set(bf2)

    outa, outb = _tail(adjp, xp, sts, fro, ent_sum, wp2p, asp2, adp2, aep2,
                       wf1g, bf1p, wf2p, bf2p)
    return outa[0:1, 0:2], outb[0, 0]


# speedup vs baseline: 17.3378x; 1.1896x over previous
import functools

import jax
import jax.numpy as jnp
from jax import lax
from jax.experimental import pallas as pl
from jax.experimental.pallas import tpu as pltpu
from jax.experimental.pallas import tpu_sc as plsc

N_PAD = 10240
E_PAD = 163840
BN = 1024
BE = 2048
ADJ_BR = 400
ADJ_BC = 1024


def _node_prep_body(x_ref, w_ref, asrc_ref, adst_ref, h_ref, ss_ref, sd_ref):
    h = jnp.dot(x_ref[...], w_ref[...], preferred_element_type=jnp.float32)
    h_ref[...] = h
    ss_ref[...] = jnp.dot(h, asrc_ref[...], preferred_element_type=jnp.float32)
    sd_ref[...] = jnp.dot(h, adst_ref[...], preferred_element_type=jnp.float32)


def _node_prep(xp, w, asrc, adst):
    npad, din = xp.shape
    dh = w.shape[1]
    grid = npad // BN
    return pl.pallas_call(
        _node_prep_body,
        grid=(grid,),
        in_specs=[
            pl.BlockSpec((BN, din), lambda i: (i, 0)),
            pl.BlockSpec((din, dh), lambda i: (0, 0)),
            pl.BlockSpec((dh, 32), lambda i: (0, 0)),
            pl.BlockSpec((dh, 32), lambda i: (0, 0)),
        ],
        out_specs=[
            pl.BlockSpec((BN, dh), lambda i: (i, 0)),
            pl.BlockSpec((BN, 32), lambda i: (i, 0)),
            pl.BlockSpec((BN, 32), lambda i: (i, 0)),
        ],
        out_shape=[
            jax.ShapeDtypeStruct((npad, dh), jnp.float32),
            jax.ShapeDtypeStruct((npad, 32), jnp.float32),
            jax.ShapeDtypeStruct((npad, 32), jnp.float32),
        ],
    )(xp, w, asrc, adst)


def _edge_prep_body(ea_ref, ae1_ref, ae2_ref, et1_ref, et2_ref):
    ea = ea_ref[...]
    et1_ref[...] = jnp.dot(ea, ae1_ref[...], preferred_element_type=jnp.float32)
    et2_ref[...] = jnp.dot(ea, ae2_ref[...], preferred_element_type=jnp.float32)


def _edge_prep(eap, ae1, ae2):
    epad, de = eap.shape
    grid = epad // BE
    return pl.pallas_call(
        _edge_prep_body,
        grid=(grid,),
        in_specs=[
            pl.BlockSpec((BE, de), lambda i: (i, 0)),
            pl.BlockSpec((de, 32), lambda i: (0, 0)),
            pl.BlockSpec((de, 32), lambda i: (0, 0)),
        ],
        out_specs=[
            pl.BlockSpec((BE, 32), lambda i: (i, 0)),
            pl.BlockSpec((BE, 32), lambda i: (i, 0)),
        ],
        out_shape=[
            jax.ShapeDtypeStruct((epad, 32), jnp.float32),
            jax.ShapeDtypeStruct((epad, 32), jnp.float32),
        ],
    )(eap, ae1, ae2)


def _maxes_body(ss_ref, sd_ref, et_ref, k_ref, a1_ref, a2_ref, a3_ref, *, ng):
    i = pl.program_id(0)

    @pl.when(i == 0)
    def _():
        a1_ref[...] = jnp.full_like(a1_ref, -1e30)
        a2_ref[...] = jnp.full_like(a2_ref, -1e30)
        a3_ref[...] = jnp.full_like(a3_ref, -1e30)

    def colmax(r):
        return jnp.broadcast_to(jnp.max(r[...], axis=0)[None, :], (8, 32))

    a1_ref[...] = jnp.maximum(a1_ref[...], colmax(ss_ref))
    a2_ref[...] = jnp.maximum(a2_ref[...], colmax(sd_ref))
    a3_ref[...] = jnp.maximum(a3_ref[...], colmax(et_ref))

    @pl.when(i == ng - 1)
    def _():
        m = a1_ref[...] + a2_ref[...] + a3_ref[...]
        k_ref[...] = jnp.maximum(m, 0.2 * m)


def _maxes(ss, sd, et):
    npad = ss.shape[0]
    epad = et.shape[0]
    nb = npad // BN
    ng = epad // BE
    return pl.pallas_call(
        functools.partial(_maxes_body, ng=ng),
        grid=(ng,),
        in_specs=[
            pl.BlockSpec((BN, 32), lambda i: (i % nb, 0)),
            pl.BlockSpec((BN, 32), lambda i: (i % nb, 0)),
            pl.BlockSpec((BE, 32), lambda i: (i, 0)),
        ],
        out_specs=pl.BlockSpec((8, 32), lambda i: (0, 0)),
        out_shape=jax.ShapeDtypeStruct((8, 32), jnp.float32),
        scratch_shapes=[
            pltpu.VMEM((8, 32), jnp.float32),
            pltpu.VMEM((8, 32), jnp.float32),
            pltpu.VMEM((8, 32), jnp.float32),
        ],
    )(ss, sd, et)


def _rden_body(d_ref, r_ref):
    d = d_ref[0] + d_ref[1]
    r_ref[...] = 1.0 / (d + 1e-16)


def _rden(dparts):
    npad = dparts.shape[1]
    return pl.pallas_call(
        _rden_body,
        out_shape=jax.ShapeDtypeStruct((npad, 32), jnp.float32),
    )(dparts)


def _prep2_body(a_ref, w_ref, asrc_ref, adst_ref, x1_ref, h2_ref, ss_ref,
                sd_ref):
    x1 = a_ref[0] + a_ref[1]
    x1_ref[...] = x1
    h2 = jnp.dot(x1, w_ref[...], preferred_element_type=jnp.float32)
    h2_ref[...] = h2
    ss_ref[...] = jnp.dot(h2, asrc_ref[...], preferred_element_type=jnp.float32)
    sd_ref[...] = jnp.dot(h2, adst_ref[...], preferred_element_type=jnp.float32)


def _prep2(aggparts, wp, asrc, adst):
    npad = aggparts.shape[1]
    grid = npad // BN
    return pl.pallas_call(
        _prep2_body,
        grid=(grid,),
        in_specs=[
            pl.BlockSpec((2, BN, 32), lambda i: (0, i, 0)),
            pl.BlockSpec((32, 32), lambda i: (0, 0)),
            pl.BlockSpec((32, 32), lambda i: (0, 0)),
            pl.BlockSpec((32, 32), lambda i: (0, 0)),
        ],
        out_specs=[
            pl.BlockSpec((BN, 32), lambda i: (i, 0)),
            pl.BlockSpec((BN, 32), lambda i: (i, 0)),
            pl.BlockSpec((BN, 32), lambda i: (i, 0)),
            pl.BlockSpec((BN, 32), lambda i: (i, 0)),
        ],
        out_shape=[
            jax.ShapeDtypeStruct((npad, 32), jnp.float32),
            jax.ShapeDtypeStruct((npad, 32), jnp.float32),
            jax.ShapeDtypeStruct((npad, 32), jnp.float32),
            jax.ShapeDtypeStruct((npad, 32), jnp.float32),
        ],
    )(aggparts, wp, asrc, adst)


def _smax_body(s1_ref, s_ref, ent_ref, *, nreal):
    i = pl.program_id(0)
    z = s1_ref[0] + s1_ref[1]
    m = jnp.max(z, axis=1, keepdims=True)
    e = jnp.exp(z - m)
    sm = e / jnp.sum(e, axis=1, keepdims=True)
    rid = i * BN + lax.broadcasted_iota(jnp.int32, sm.shape, 0)
    sm = jnp.where(rid < nreal, sm, 0.0)
    s_ref[...] = sm
    ent = -jnp.sum(sm * jnp.log(sm + 1e-15))

    @pl.when(i == 0)
    def _():
        ent_ref[0, 0] = 0.0

    ent_ref[0, 0] += ent


def _smax(s1parts, nreal):
    npad = s1parts.shape[1]
    grid = npad // BN
    return pl.pallas_call(
        functools.partial(_smax_body, nreal=nreal),
        grid=(grid,),
        in_specs=[pl.BlockSpec((2, BN, 32), lambda i: (0, i, 0))],
        out_specs=[
            pl.BlockSpec((BN, 32), lambda i: (i, 0)),
            pl.BlockSpec((1, 1), lambda i: (0, 0),
                         memory_space=pltpu.SMEM),
        ],
        out_shape=[
            jax.ShapeDtypeStruct((npad, 32), jnp.float32),
            jax.ShapeDtypeStruct((1, 1), jnp.float32),
        ],
    )(s1parts)


def _adj_body(adj_ref, sk_ref, si_ref, x1_ref, adjp_ref, xp_ref, sts_ref,
              fro_ref, tmp_ref, *, nk, ncols):
    i = pl.program_id(0)
    k = pl.program_id(1)

    blk = adj_ref[...]
    colid = k * ADJ_BC + lax.broadcasted_iota(jnp.int32, blk.shape, 1)
    blk = jnp.where(colid < ncols, blk, 0.0)

    @pl.when(jnp.logical_and(i == 0, k == 0))
    def _():
        adjp_ref[...] = jnp.zeros_like(adjp_ref)
        xp_ref[...] = jnp.zeros_like(xp_ref)
        sts_ref[...] = jnp.zeros_like(sts_ref)
        fro_ref[0, 0] = 0.0

    fro_ref[0, 0] += jnp.sum(blk * blk)

    part = jnp.dot(blk, sk_ref[...], preferred_element_type=jnp.float32)

    @pl.when(k == 0)
    def _():
        tmp_ref[...] = part
        si = si_ref[...]
        xp_ref[...] += lax.dot_general(
            si, x1_ref[...], (((0,), (0,)), ((), ())),
            preferred_element_type=jnp.float32)
        sts_ref[...] += lax.dot_general(
            si, si, (((0,), (0,)), ((), ())),
            preferred_element_type=jnp.float32)

    @pl.when(k > 0)
    def _():
        tmp_ref[...] += part

    @pl.when(k == nk - 1)
    def _():
        adjp_ref[...] += lax.dot_general(
            si_ref[...], tmp_ref[...], (((0,), (0,)), ((), ())),
            preferred_element_type=jnp.float32)


def _adj_pass(adj, s, x1):
    nrows, ncols = adj.shape
    ni = nrows // ADJ_BR
    nk = (ncols + ADJ_BC - 1) // ADJ_BC
    return pl.pallas_call(
        functools.partial(_adj_body, nk=nk, ncols=ncols),
        grid=(ni, nk),
        in_specs=[
            pl.BlockSpec((ADJ_BR, ADJ_BC), lambda i, k: (i, k)),
            pl.BlockSpec((ADJ_BC, 32), lambda i, k: (k, 0)),
            pl.BlockSpec((ADJ_BR, 32), lambda i, k: (i, 0)),
            pl.BlockSpec((ADJ_BR, 32), lambda i, k: (i, 0)),
        ],
        out_specs=[
            pl.BlockSpec((32, 32), lambda i, k: (0, 0)),
            pl.BlockSpec((32, 32), lambda i, k: (0, 0)),
            pl.BlockSpec((32, 32), lambda i, k: (0, 0)),
            pl.BlockSpec((1, 1), lambda i, k: (0, 0),
                         memory_space=pltpu.SMEM),
        ],
        out_shape=[
            jax.ShapeDtypeStruct((32, 32), jnp.float32),
            jax.ShapeDtypeStruct((32, 32), jnp.float32),
            jax.ShapeDtypeStruct((32, 32), jnp.float32),
            jax.ShapeDtypeStruct((1, 1), jnp.float32),
        ],
        scratch_shapes=[pltpu.VMEM((ADJ_BR, 32), jnp.float32)],
        compiler_params=pltpu.CompilerParams(
            dimension_semantics=("arbitrary", "arbitrary")),
    )(adj, s, s, x1)


def _tail_body(adjp_ref, xp_ref, sts_ref, fro_ref, ent_ref, wp2_ref, asp2_ref,
               adp2_ref, aep2_ref, wf1_ref, bf1_ref, wf2_ref, bf2_ref,
               outa_ref, outb_ref):
    adjp = adjp_ref[...]
    x2 = xp_ref[...]
    h3 = jnp.dot(x2, wp2_ref[...], preferred_element_type=jnp.float32)
    ss3 = jnp.dot(h3, asp2_ref[...], preferred_element_type=jnp.float32)
    sd3m = lax.dot_general(adp2_ref[...], h3, (((0,), (1,)), ((), ())),
                           preferred_element_type=jnp.float32)
    alpha = ss3[:, 0:1] + sd3m[0:1, :] + adjp * aep2_ref[0, 0]
    alpha = jnp.maximum(alpha, 0.2 * alpha)
    cmax = jnp.max(alpha, axis=0, keepdims=True)
    ex = jnp.exp(alpha - cmax)
    att = ex / (jnp.sum(ex, axis=0, keepdims=True) + 1e-16)
    s2 = lax.dot_general(att, h3, (((0,), (0,)), ((), ())),
                         preferred_element_type=jnp.float32)

    colmask = lax.broadcasted_iota(jnp.int32, s2.shape, 1) < 4
    z = jnp.where(colmask, s2, -1e30)
    m2 = jnp.max(z, axis=1, keepdims=True)
    e2 = jnp.where(colmask, jnp.exp(z - m2), 0.0)
    s2s = e2 / jnp.sum(e2, axis=1, keepdims=True)
    ent2 = -jnp.sum(s2s * jnp.log(s2s + 1e-15)) / 32.0

    x3 = lax.dot_general(s2s, x2, (((0,), (0,)), ((), ())),
                         preferred_element_type=jnp.float32)
    adjs2 = jnp.dot(adjp, s2s, preferred_element_type=jnp.float32)
    adjp2 = lax.dot_general(s2s, adjs2, (((0,), (0,)), ((), ())),
                            preferred_element_type=jnp.float32)
    sts2 = lax.dot_general(s2s, s2s, (((0,), (0,)), ((), ())),
                           preferred_element_type=jnp.float32)
    eye8 = (lax.broadcasted_iota(jnp.int32, (8, 8), 0)
            == lax.broadcasted_iota(jnp.int32, (8, 8), 1))
    tr2 = jnp.sum(jnp.where(eye8, adjp2, 0.0))
    link2sq = jnp.sum(adjp * adjp) - 2.0 * tr2 + jnp.sum(sts2 * sts2)
    link2 = jnp.sqrt(jnp.maximum(link2sq, 1e-12)) / 32.0
    reg2 = link2 + ent2

    sts1 = sts_ref[...]
    eye32 = (lax.broadcasted_iota(jnp.int32, (32, 32), 0)
             == lax.broadcasted_iota(jnp.int32, (32, 32), 1))
    tr1 = jnp.sum(jnp.where(eye32, adjp, 0.0))
    link1sq = fro_ref[0, 0] - 2.0 * tr1 + jnp.sum(sts1 * sts1)
    link1 = jnp.sqrt(jnp.maximum(link1sq, 1e-12)) / 10000.0
    ent1 = ent_ref[0, 0] / 10000.0
    reg = (link1 + ent1) * 10.0 + reg2 * 0.1

    acc = bf1_ref[...]
    for r in range(8):
        acc = acc + jnp.dot(x3[r:r + 1, :], wf1_ref[r],
                            preferred_element_type=jnp.float32)
    h1f = jnp.maximum(acc, 0.0)
    out2 = jnp.dot(h1f, wf2_ref[...], preferred_element_type=jnp.float32) \
        + bf2_ref[...]
    outa_ref[...] = jnp.broadcast_to(out2, (8, 128))
    outb_ref[...] = jnp.full((8, 128), reg)


def _tail(adjp, xp, sts, fro, ent, wp2, asp2, adp2, aep2, wf1g, bf1p, wf2p,
          bf2p):
    vm = pl.BlockSpec(memory_space=pltpu.VMEM)
    sm = pl.BlockSpec(memory_space=pltpu.SMEM)
    return pl.pallas_call(
        _tail_body,
        in_specs=[vm, vm, vm, sm, sm, vm, vm, vm, sm, vm, vm, vm, vm],
        out_shape=[
            jax.ShapeDtypeStruct((8, 128), jnp.float32),
            jax.ShapeDtypeStruct((8, 128), jnp.float32),
        ],
    )(adjp, xp, sts, fro, ent, wp2, asp2, adp2, aep2, wf1g, bf1p, wf2p, bf2p)


NW = 32
EC = E_PAD // NW
NCH = EC // 128
NROWS = N_PAD // 16


def _sc_mesh():
    return plsc.VectorSubcoreMesh(core_axis_name="c", subcore_axis_name="s")


def _sc_phase_a(ss, sd, et, k32, src2d, dst2d):
    @functools.partial(
        pl.kernel,
        out_type=[
            jax.ShapeDtypeStruct((2, N_PAD, 32), jnp.float32),
            jax.ShapeDtypeStruct((E_PAD, 32), jnp.float32),
        ],
        mesh=_sc_mesh(),
        compiler_params=pltpu.CompilerParams(use_tc_tiling_on_sc=False),
        scratch_types=[
            pltpu.VMEM((NCH, 128), jnp.int32),
            pltpu.VMEM((NCH, 128), jnp.int32),
            pltpu.VMEM((128, 32), jnp.float32),
            pltpu.VMEM((128, 32), jnp.float32),
            pltpu.VMEM((128, 32), jnp.float32),
            pltpu.VMEM((128, 32), jnp.float32),
            pltpu.VMEM((128, 32), jnp.float32),
            pltpu.VMEM((128, 32), jnp.float32),
            pltpu.VMEM((128, 32), jnp.float32),
            pltpu.VMEM((128, 32), jnp.float32),
            pltpu.VMEM((32,), jnp.float32),
            pltpu.VMEM((NROWS, 32), jnp.float32),
            pltpu.VMEM_SHARED((N_PAD, 32), jnp.float32),
            pltpu.SemaphoreType.DMA,
            pltpu.SemaphoreType.DMA,
            pltpu.SemaphoreType.DMA,
            pltpu.SemaphoreType.DMA,
            pltpu.SemaphoreType.DMA,
            pltpu.SemaphoreType.DMA,
            pltpu.SemaphoreType.DMA,
            pltpu.SemaphoreType.DMA,
        ],
    )
    def k(ss_hbm, sd_hbm, et_hbm, k_hbm, src_hbm, dst_hbm, den_out, ex_out,
          srci, dsti, ssb0, sdb0, etb0, exb0, ssb1, sdb1, etb1, exb1,
          kv, stage, den_sh, gsem0, gsem1, lsem0, lsem1, ssem0, ssem1,
          wsem0, wsem1):
        cid = lax.axis_index("c")
        sid = lax.axis_index("s")
        wid = sid * 2 + cid
        base = wid * EC

        pltpu.sync_copy(k_hbm, kv)
        pltpu.sync_copy(src_hbm.at[pl.ds(wid * NCH, NCH), :], srci)
        pltpu.sync_copy(dst_hbm.at[pl.ds(wid * NCH, NCH), :], dsti)
        k0 = kv[0:16]
        k1 = kv[16:32]

        def zbody(i, _):
            stage[i, 0:16] = jnp.zeros((16,), jnp.float32)
            stage[i, 16:32] = jnp.zeros((16,), jnp.float32)
            return 0

        lax.fori_loop(0, NROWS, zbody, 0)
        pltpu.sync_copy(stage, den_sh.at[pl.ds(sid * NROWS, NROWS)])
        plsc.subcore_barrier()

        def issue_in(r, ssb, sdb, etb, gsem, lsem):
            off = base + r * 128
            pltpu.async_copy(ss_hbm.at[srci.at[r]], ssb, gsem)
            pltpu.async_copy(sd_hbm.at[dsti.at[r]], sdb, gsem)
            pltpu.async_copy(et_hbm.at[pl.ds(off, 128), :], etb, lsem)

        def wait_in(r, ssb, sdb, etb, gsem, lsem):
            off = base + r * 128
            pltpu.make_async_copy(ss_hbm.at[srci.at[r]], ssb, gsem).wait()
            pltpu.make_async_copy(sd_hbm.at[dsti.at[r]], sdb, gsem).wait()
            pltpu.make_async_copy(
                et_hbm.at[pl.ds(off, 128), :], etb, lsem).wait()

        def compute(ssb, sdb, etb, exb):
            def jbody(j, _):
                a0 = ssb[j, 0:16] + sdb[j, 0:16] + etb[j, 0:16]
                a0 = jnp.maximum(a0, 0.2 * a0) - k0
                exb[j, 0:16] = jnp.exp(a0)
                a1 = ssb[j, 16:32] + sdb[j, 16:32] + etb[j, 16:32]
                a1 = jnp.maximum(a1, 0.2 * a1) - k1
                exb[j, 16:32] = jnp.exp(a1)
                return 0

            lax.fori_loop(0, 128, jbody, 0, unroll=4)

        def issue_out(r, exb, ssem, wsem):
            off = base + r * 128
            pltpu.async_copy(exb, den_sh.at[dsti.at[r]], ssem, add=True)
            pltpu.async_copy(exb, ex_out.at[pl.ds(off, 128), :], wsem)

        def wait_out(r, exb, ssem, wsem):
            off = base + r * 128
            pltpu.make_async_copy(
                exb, den_sh.at[dsti.at[r]], ssem).wait()
            pltpu.make_async_copy(
                exb, ex_out.at[pl.ds(off, 128), :], wsem).wait()

        issue_in(0, ssb0, sdb0, etb0, gsem0, lsem0)
        issue_in(1, ssb1, sdb1, etb1, gsem1, lsem1)

        def half(c, r, ssb, sdb, etb, exb, gsem, lsem, ssem, wsem):
            wait_in(r, ssb, sdb, etb, gsem, lsem)

            @pl.when(c > 0)
            def _():
                wait_out(r - 2, exb, ssem, wsem)

            compute(ssb, sdb, etb, exb)
            issue_out(r, exb, ssem, wsem)

            @pl.when(c < NCH // 2 - 1)
            def _():
                issue_in(r + 2, ssb, sdb, etb, gsem, lsem)

        def chunk2(c, _):
            half(c, 2 * c, ssb0, sdb0, etb0, exb0, gsem0, lsem0, ssem0,
                 wsem0)
            half(c, 2 * c + 1, ssb1, sdb1, etb1, exb1, gsem1, lsem1, ssem1,
                 wsem1)
            return 0

        lax.fori_loop(0, NCH // 2, chunk2, 0)
        wait_out(NCH - 2, exb0, ssem0, wsem0)
        wait_out(NCH - 1, exb1, ssem1, wsem1)
        plsc.subcore_barrier()
        pltpu.sync_copy(
            den_sh.at[pl.ds(sid * NROWS, NROWS)],
            den_out.at[cid, pl.ds(sid * NROWS, NROWS), :])

    return k(ss, sd, et, k32, src2d, dst2d)


def _sc_phase_b(ex, rden, h, src2d, dst2d):
    @functools.partial(
        pl.kernel,
        out_type=jax.ShapeDtypeStruct((2, N_PAD, 32), jnp.float32),
        mesh=_sc_mesh(),
        compiler_params=pltpu.CompilerParams(use_tc_tiling_on_sc=False),
        scratch_types=[
            pltpu.VMEM((NCH, 128), jnp.int32),
            pltpu.VMEM((NCH, 128), jnp.int32),
            pltpu.VMEM((128, 32), jnp.float32),
            pltpu.VMEM((128, 32), jnp.float32),
            pltpu.VMEM((128, 32), jnp.float32),
            pltpu.VMEM((128, 32), jnp.float32),
            pltpu.VMEM((128, 32), jnp.float32),
            pltpu.VMEM((128, 32), jnp.float32),
            pltpu.VMEM((128, 32), jnp.float32),
            pltpu.VMEM((128, 32), jnp.float32),
            pltpu.VMEM((NROWS, 32), jnp.float32),
            pltpu.VMEM_SHARED((N_PAD, 32), jnp.float32),
            pltpu.SemaphoreType.DMA,
            pltpu.SemaphoreType.DMA,
            pltpu.SemaphoreType.DMA,
            pltpu.SemaphoreType.DMA,
            pltpu.SemaphoreType.DMA,
            pltpu.SemaphoreType.DMA,
        ],
    )
    def k(ex_hbm, rd_hbm, h_hbm, src_hbm, dst_hbm, agg_out,
          srci, dsti, exb0, rdb0, hb0, msgb0, exb1, rdb1, hb1, msgb1,
          stage, agg_sh, gsem0, gsem1, lsem0, lsem1, ssem0, ssem1):
        cid = lax.axis_index("c")
        sid = lax.axis_index("s")
        wid = sid * 2 + cid
        base = wid * EC

        pltpu.sync_copy(src_hbm.at[pl.ds(wid * NCH, NCH), :], srci)
        pltpu.sync_copy(dst_hbm.at[pl.ds(wid * NCH, NCH), :], dsti)

        def zbody(i, _):
            stage[i, 0:16] = jnp.zeros((16,), jnp.float32)
            stage[i, 16:32] = jnp.zeros((16,), jnp.float32)
            return 0

        lax.fori_loop(0, NROWS, zbody, 0)
        pltpu.sync_copy(stage, agg_sh.at[pl.ds(sid * NROWS, NROWS)])
        plsc.subcore_barrier()

        def issue_in(r, rdb, hb, exb, gsem, lsem):
            off = base + r * 128
            pltpu.async_copy(rd_hbm.at[dsti.at[r]], rdb, gsem)
            pltpu.async_copy(h_hbm.at[srci.at[r]], hb, gsem)
            pltpu.async_copy(ex_hbm.at[pl.ds(off, 128), :], exb, lsem)

        def wait_in(r, rdb, hb, exb, gsem, lsem):
            off = base + r * 128
            pltpu.make_async_copy(rd_hbm.at[dsti.at[r]], rdb, gsem).wait()
            pltpu.make_async_copy(h_hbm.at[srci.at[r]], hb, gsem).wait()
            pltpu.make_async_copy(
                ex_hbm.at[pl.ds(off, 128), :], exb, lsem).wait()

        def compute(rdb, hb, exb, msgb):
            def jbody(j, _):
                msgb[j, 0:16] = hb[j, 0:16] * exb[j, 0:16] * rdb[j, 0:16]
                msgb[j, 16:32] = hb[j, 16:32] * exb[j, 16:32] * rdb[j, 16:32]
                return 0

            lax.fori_loop(0, 128, jbody, 0, unroll=4)

        issue_in(0, rdb0, hb0, exb0, gsem0, lsem0)
        issue_in(1, rdb1, hb1, exb1, gsem1, lsem1)

        def half(c, r, rdb, hb, exb, msgb, gsem, lsem, ssem):
            wait_in(r, rdb, hb, exb, gsem, lsem)

            @pl.when(c > 0)
            def _():
                pltpu.make_async_copy(
                    msgb, agg_sh.at[dsti.at[r - 2]], ssem).wait()

            compute(rdb, hb, exb, msgb)
            pltpu.async_copy(msgb, agg_sh.at[dsti.at[r]], ssem, add=True)

            @pl.when(c < NCH // 2 - 1)
            def _():
                issue_in(r + 2, rdb, hb, exb, gsem, lsem)

        def chunk2(c, _):
            half(c, 2 * c, rdb0, hb0, exb0, msgb0, gsem0, lsem0, ssem0)
            half(c, 2 * c + 1, rdb1, hb1, exb1, msgb1, gsem1, lsem1, ssem1)
            return 0

        lax.fori_loop(0, NCH // 2, chunk2, 0)
        pltpu.make_async_copy(
            msgb0, agg_sh.at[dsti.at[NCH - 2]], ssem0).wait()
        pltpu.make_async_copy(
            msgb1, agg_sh.at[dsti.at[NCH - 1]], ssem1).wait()
        plsc.subcore_barrier()
        pltpu.sync_copy(
            agg_sh.at[pl.ds(sid * NROWS, NROWS)],
            agg_out.at[cid, pl.ds(sid * NROWS, NROWS), :])

    return k(ex, rden, h, src2d, dst2d)


def _conv_segops(ss, sd, et, k32, h, srcp, dstp):
    dparts, ex = _sc_phase_a(ss, sd, et, k32, srcp, dstp)
    rden = _rden(dparts)
    return _sc_phase_b(ex, rden, h, srcp, dstp)


def kernel(x, edge_index, edge_attr, y, adj, W1, a_src1, a_dst1, a_edge1,
           Wp1, a_src_p1, a_dst_p1, a_edge_p1, Wp2, a_src_p2, a_dst_p2,
           a_edge_p2, Wf1, bf1, Wf2, bf2):
    n, dfeat = x.shape
    e = edge_index.shape[1]

    xp_in = jnp.pad(x, ((0, N_PAD - n), (0, 0)))
    srcp = jnp.concatenate(
        [edge_index[0].astype(jnp.int32),
         jnp.zeros((E_PAD - e,), jnp.int32)]).reshape(E_PAD // 128, 128)
    dstp = jnp.concatenate(
        [edge_index[1].astype(jnp.int32),
         jnp.full((E_PAD - e,), n, jnp.int32)]).reshape(E_PAD // 128, 128)
    eap = jnp.pad(edge_attr, ((0, E_PAD - e), (0, 0)))

    w1p = jnp.pad(W1, ((0, 0), (0, 2)))
    asrc1 = jnp.zeros((32, 32), jnp.float32)
    adst1 = jnp.zeros((32, 32), jnp.float32)
    ae1p = jnp.zeros((4, 32), jnp.float32)
    for hh in range(5):
        blk_s = jnp.broadcast_to(a_src1[hh][:, None], (6, 6))
        blk_d = jnp.broadcast_to(a_dst1[hh][:, None], (6, 6))
        asrc1 = asrc1.at[hh * 6:(hh + 1) * 6, hh * 6:(hh + 1) * 6].set(blk_s)
        adst1 = adst1.at[hh * 6:(hh + 1) * 6, hh * 6:(hh + 1) * 6].set(blk_d)
        ae1p = ae1p.at[:, hh * 6:(hh + 1) * 6].set(
            jnp.broadcast_to(a_edge1[:, hh:hh + 1], (4, 6)))

    wp1p = jnp.pad(Wp1, ((0, 2), (0, 0)))
    asrc2 = jnp.broadcast_to(a_src_p1[0][:, None], (32, 32))
    adst2 = jnp.broadcast_to(a_dst_p1[0][:, None], (32, 32))
    ae2p = jnp.broadcast_to(a_edge_p1, (4, 32))

    h1p, ss1, sd1 = _node_prep(xp_in, w1p, asrc1, adst1)
    et1, et2 = _edge_prep(eap, ae1p, ae2p)
    k1 = _maxes(ss1, sd1, et1)[0]

    aggparts1 = _conv_segops(ss1, sd1, et1, k1, h1p, srcp, dstp)

    x1, h2p, ss2, sd2 = _prep2(aggparts1, wp1p, asrc2, adst2)
    k2 = _maxes(ss2, sd2, et2)[0]

    aggparts2 = _conv_segops(ss2, sd2, et2, k2, h2p, srcp, dstp)

    s, ent_sum = _smax(aggparts2, n)
    adjp, xp, sts, fro = _adj_pass(adj, s, x1)

    wp2p = jnp.pad(Wp2, ((0, 2), (0, 4)))
    asp2 = jnp.zeros((8, 8), jnp.float32).at[0:4, 0].set(a_src_p2[0])
    adp2 = jnp.zeros((8, 8), jnp.float32).at[0:4, 0].set(a_dst_p2[0])
    aep2 = a_edge_p2.reshape(1, 1)
    wf1g = jnp.zeros((8, 32, 32), jnp.float32).at[0:4, 0:30, :].set(
        Wf1.reshape(4, 30, 32))
    bf1p = bf1.reshape(1, 32)
    wf2p = jnp.zeros((32, 128), jnp.float32).at[:, 0:2].set(Wf2)
    bf2p = jnp.zeros((1, 128), jnp.float32).at[0, 0:2].set(bf2)

    outa, outb = _tail(adjp, xp, sts, fro, ent_sum, wp2p, asp2, adp2, aep2,
                       wf1g, bf1p, wf2p, bf2p)
    return outa[0:1, 0:2], outb[0, 0]


# speedup vs baseline: 19.1162x; 1.1026x over previous
import functools

import jax
import jax.numpy as jnp
from jax import lax
from jax.experimental import pallas as pl
from jax.experimental.pallas import tpu as pltpu
from jax.experimental.pallas import tpu_sc as plsc

N_PAD = 10240
E_PAD = 163840
BN = 1024
BE = 2048
ADJ_BR = 1000
ADJ_BC = 2048


def _node_prep_body(x_ref, w_ref, asrc_ref, adst_ref, h_ref, ss_ref, sd_ref):
    h = jnp.dot(x_ref[...], w_ref[...], preferred_element_type=jnp.float32)
    h_ref[...] = h
    ss_ref[...] = jnp.dot(h, asrc_ref[...], preferred_element_type=jnp.float32)
    sd_ref[...] = jnp.dot(h, adst_ref[...], preferred_element_type=jnp.float32)


def _node_prep(xp, w, asrc, adst):
    npad, din = xp.shape
    dh = w.shape[1]
    grid = npad // BN
    return pl.pallas_call(
        _node_prep_body,
        grid=(grid,),
        in_specs=[
            pl.BlockSpec((BN, din), lambda i: (i, 0)),
            pl.BlockSpec((din, dh), lambda i: (0, 0)),
            pl.BlockSpec((dh, 32), lambda i: (0, 0)),
            pl.BlockSpec((dh, 32), lambda i: (0, 0)),
        ],
        out_specs=[
            pl.BlockSpec((BN, dh), lambda i: (i, 0)),
            pl.BlockSpec((BN, 32), lambda i: (i, 0)),
            pl.BlockSpec((BN, 32), lambda i: (i, 0)),
        ],
        out_shape=[
            jax.ShapeDtypeStruct((npad, dh), jnp.float32),
            jax.ShapeDtypeStruct((npad, 32), jnp.float32),
            jax.ShapeDtypeStruct((npad, 32), jnp.float32),
        ],
    )(xp, w, asrc, adst)


def _edge_prep_body(ea_ref, ae1_ref, ae2_ref, et1_ref, et2_ref):
    ea = ea_ref[...]
    et1_ref[...] = jnp.dot(ea, ae1_ref[...], preferred_element_type=jnp.float32)
    et2_ref[...] = jnp.dot(ea, ae2_ref[...], preferred_element_type=jnp.float32)


def _edge_prep(eap, ae1, ae2):
    epad, de = eap.shape
    grid = epad // BE
    return pl.pallas_call(
        _edge_prep_body,
        grid=(grid,),
        in_specs=[
            pl.BlockSpec((BE, de), lambda i: (i, 0)),
            pl.BlockSpec((de, 32), lambda i: (0, 0)),
            pl.BlockSpec((de, 32), lambda i: (0, 0)),
        ],
        out_specs=[
            pl.BlockSpec((BE, 32), lambda i: (i, 0)),
            pl.BlockSpec((BE, 32), lambda i: (i, 0)),
        ],
        out_shape=[
            jax.ShapeDtypeStruct((epad, 32), jnp.float32),
            jax.ShapeDtypeStruct((epad, 32), jnp.float32),
        ],
    )(eap, ae1, ae2)


def _maxes_body(ss_ref, sd_ref, et_ref, k_ref, a1_ref, a2_ref, a3_ref, *, ng):
    i = pl.program_id(0)

    @pl.when(i == 0)
    def _():
        a1_ref[...] = jnp.full_like(a1_ref, -1e30)
        a2_ref[...] = jnp.full_like(a2_ref, -1e30)
        a3_ref[...] = jnp.full_like(a3_ref, -1e30)

    def colmax(r):
        return jnp.broadcast_to(jnp.max(r[...], axis=0)[None, :], (8, 32))

    a1_ref[...] = jnp.maximum(a1_ref[...], colmax(ss_ref))
    a2_ref[...] = jnp.maximum(a2_ref[...], colmax(sd_ref))
    a3_ref[...] = jnp.maximum(a3_ref[...], colmax(et_ref))

    @pl.when(i == ng - 1)
    def _():
        m = a1_ref[...] + a2_ref[...] + a3_ref[...]
        k_ref[...] = jnp.maximum(m, 0.2 * m)


def _maxes(ss, sd, et):
    npad = ss.shape[0]
    epad = et.shape[0]
    nb = npad // BN
    ng = epad // BE
    return pl.pallas_call(
        functools.partial(_maxes_body, ng=ng),
        grid=(ng,),
        in_specs=[
            pl.BlockSpec((BN, 32), lambda i: (i % nb, 0)),
            pl.BlockSpec((BN, 32), lambda i: (i % nb, 0)),
            pl.BlockSpec((BE, 32), lambda i: (i, 0)),
        ],
        out_specs=pl.BlockSpec((8, 32), lambda i: (0, 0)),
        out_shape=jax.ShapeDtypeStruct((8, 32), jnp.float32),
        scratch_shapes=[
            pltpu.VMEM((8, 32), jnp.float32),
            pltpu.VMEM((8, 32), jnp.float32),
            pltpu.VMEM((8, 32), jnp.float32),
        ],
    )(ss, sd, et)


def _rden_body(d_ref, r_ref):
    d = d_ref[0] + d_ref[1]
    r_ref[...] = 1.0 / (d + 1e-16)


def _rden(dparts):
    npad = dparts.shape[1]
    return pl.pallas_call(
        _rden_body,
        out_shape=jax.ShapeDtypeStruct((npad, 32), jnp.float32),
    )(dparts)


def _prep2_body(a_ref, w_ref, asrc_ref, adst_ref, x1_ref, h2_ref, ss_ref,
                sd_ref):
    x1 = a_ref[0] + a_ref[1]
    x1_ref[...] = x1
    h2 = jnp.dot(x1, w_ref[...], preferred_element_type=jnp.float32)
    h2_ref[...] = h2
    ss_ref[...] = jnp.dot(h2, asrc_ref[...], preferred_element_type=jnp.float32)
    sd_ref[...] = jnp.dot(h2, adst_ref[...], preferred_element_type=jnp.float32)


def _prep2(aggparts, wp, asrc, adst):
    npad = aggparts.shape[1]
    grid = npad // BN
    return pl.pallas_call(
        _prep2_body,
        grid=(grid,),
        in_specs=[
            pl.BlockSpec((2, BN, 32), lambda i: (0, i, 0)),
            pl.BlockSpec((32, 32), lambda i: (0, 0)),
            pl.BlockSpec((32, 32), lambda i: (0, 0)),
            pl.BlockSpec((32, 32), lambda i: (0, 0)),
        ],
        out_specs=[
            pl.BlockSpec((BN, 32), lambda i: (i, 0)),
            pl.BlockSpec((BN, 32), lambda i: (i, 0)),
            pl.BlockSpec((BN, 32), lambda i: (i, 0)),
            pl.BlockSpec((BN, 32), lambda i: (i, 0)),
        ],
        out_shape=[
            jax.ShapeDtypeStruct((npad, 32), jnp.float32),
            jax.ShapeDtypeStruct((npad, 32), jnp.float32),
            jax.ShapeDtypeStruct((npad, 32), jnp.float32),
            jax.ShapeDtypeStruct((npad, 32), jnp.float32),
        ],
    )(aggparts, wp, asrc, adst)


def _smax_body(s1_ref, s_ref, ent_ref, *, nreal):
    i = pl.program_id(0)
    z = s1_ref[0] + s1_ref[1]
    m = jnp.max(z, axis=1, keepdims=True)
    e = jnp.exp(z - m)
    sm = e / jnp.sum(e, axis=1, keepdims=True)
    rid = i * BN + lax.broadcasted_iota(jnp.int32, sm.shape, 0)
    sm = jnp.where(rid < nreal, sm, 0.0)
    s_ref[...] = sm
    ent = -jnp.sum(sm * jnp.log(sm + 1e-15))

    @pl.when(i == 0)
    def _():
        ent_ref[0, 0] = 0.0

    ent_ref[0, 0] += ent


def _smax(s1parts, nreal):
    npad = s1parts.shape[1]
    grid = npad // BN
    return pl.pallas_call(
        functools.partial(_smax_body, nreal=nreal),
        grid=(grid,),
        in_specs=[pl.BlockSpec((2, BN, 32), lambda i: (0, i, 0))],
        out_specs=[
            pl.BlockSpec((BN, 32), lambda i: (i, 0)),
            pl.BlockSpec((1, 1), lambda i: (0, 0),
                         memory_space=pltpu.SMEM),
        ],
        out_shape=[
            jax.ShapeDtypeStruct((npad, 32), jnp.float32),
            jax.ShapeDtypeStruct((1, 1), jnp.float32),
        ],
    )(s1parts)


def _adj_body(adj_ref, sk_ref, si_ref, x1_ref, adjp_ref, xp_ref, sts_ref,
              fro_ref, tmp_ref, *, nk, ncols):
    i = pl.program_id(0)
    k = pl.program_id(1)

    blk = adj_ref[...]
    colid = k * ADJ_BC + lax.broadcasted_iota(jnp.int32, blk.shape, 1)
    blk = jnp.where(colid < ncols, blk, 0.0)

    @pl.when(jnp.logical_and(i == 0, k == 0))
    def _():
        adjp_ref[...] = jnp.zeros_like(adjp_ref)
        xp_ref[...] = jnp.zeros_like(xp_ref)
        sts_ref[...] = jnp.zeros_like(sts_ref)
        fro_ref[0, 0] = 0.0

    fro_ref[0, 0] += jnp.sum(blk * blk)

    part = jnp.dot(blk, sk_ref[...], preferred_element_type=jnp.float32)

    @pl.when(k == 0)
    def _():
        tmp_ref[...] = part
        si = si_ref[...]
        xp_ref[...] += lax.dot_general(
            si, x1_ref[...], (((0,), (0,)), ((), ())),
            preferred_element_type=jnp.float32)
        sts_ref[...] += lax.dot_general(
            si, si, (((0,), (0,)), ((), ())),
            preferred_element_type=jnp.float32)

    @pl.when(k > 0)
    def _():
        tmp_ref[...] += part

    @pl.when(k == nk - 1)
    def _():
        adjp_ref[...] += lax.dot_general(
            si_ref[...], tmp_ref[...], (((0,), (0,)), ((), ())),
            preferred_element_type=jnp.float32)


def _adj_pass(adj, s, x1):
    nrows, ncols = adj.shape
    ni = nrows // ADJ_BR
    nk = (ncols + ADJ_BC - 1) // ADJ_BC
    return pl.pallas_call(
        functools.partial(_adj_body, nk=nk, ncols=ncols),
        grid=(ni, nk),
        in_specs=[
            pl.BlockSpec((ADJ_BR, ADJ_BC), lambda i, k: (i, k)),
            pl.BlockSpec((ADJ_BC, 32), lambda i, k: (k, 0)),
            pl.BlockSpec((ADJ_BR, 32), lambda i, k: (i, 0)),
            pl.BlockSpec((ADJ_BR, 32), lambda i, k: (i, 0)),
        ],
        out_specs=[
            pl.BlockSpec((32, 32), lambda i, k: (0, 0)),
            pl.BlockSpec((32, 32), lambda i, k: (0, 0)),
            pl.BlockSpec((32, 32), lambda i, k: (0, 0)),
            pl.BlockSpec((1, 1), lambda i, k: (0, 0),
                         memory_space=pltpu.SMEM),
        ],
        out_shape=[
            jax.ShapeDtypeStruct((32, 32), jnp.float32),
            jax.ShapeDtypeStruct((32, 32), jnp.float32),
            jax.ShapeDtypeStruct((32, 32), jnp.float32),
            jax.ShapeDtypeStruct((1, 1), jnp.float32),
        ],
        scratch_shapes=[pltpu.VMEM((ADJ_BR, 32), jnp.float32)],
        compiler_params=pltpu.CompilerParams(
            dimension_semantics=("arbitrary", "arbitrary")),
    )(adj, s, s, x1)


def _tail_body(adjp_ref, xp_ref, sts_ref, fro_ref, ent_ref, wp2_ref, asp2_ref,
               adp2_ref, aep2_ref, wf1_ref, bf1_ref, wf2_ref, bf2_ref,
               outa_ref, outb_ref):
    adjp = adjp_ref[...]
    x2 = xp_ref[...]
    h3 = jnp.dot(x2, wp2_ref[...], preferred_element_type=jnp.float32)
    ss3 = jnp.dot(h3, asp2_ref[...], preferred_element_type=jnp.float32)
    sd3m = lax.dot_general(adp2_ref[...], h3, (((0,), (1,)), ((), ())),
                           preferred_element_type=jnp.float32)
    alpha = ss3[:, 0:1] + sd3m[0:1, :] + adjp * aep2_ref[0, 0]
    alpha = jnp.maximum(alpha, 0.2 * alpha)
    cmax = jnp.max(alpha, axis=0, keepdims=True)
    ex = jnp.exp(alpha - cmax)
    att = ex / (jnp.sum(ex, axis=0, keepdims=True) + 1e-16)
    s2 = lax.dot_general(att, h3, (((0,), (0,)), ((), ())),
                         preferred_element_type=jnp.float32)

    colmask = lax.broadcasted_iota(jnp.int32, s2.shape, 1) < 4
    z = jnp.where(colmask, s2, -1e30)
    m2 = jnp.max(z, axis=1, keepdims=True)
    e2 = jnp.where(colmask, jnp.exp(z - m2), 0.0)
    s2s = e2 / jnp.sum(e2, axis=1, keepdims=True)
    ent2 = -jnp.sum(s2s * jnp.log(s2s + 1e-15)) / 32.0

    x3 = lax.dot_general(s2s, x2, (((0,), (0,)), ((), ())),
                         preferred_element_type=jnp.float32)
    adjs2 = jnp.dot(adjp, s2s, preferred_element_type=jnp.float32)
    adjp2 = lax.dot_general(s2s, adjs2, (((0,), (0,)), ((), ())),
                            preferred_element_type=jnp.float32)
    sts2 = lax.dot_general(s2s, s2s, (((0,), (0,)), ((), ())),
                           preferred_element_type=jnp.float32)
    eye8 = (lax.broadcasted_iota(jnp.int32, (8, 8), 0)
            == lax.broadcasted_iota(jnp.int32, (8, 8), 1))
    tr2 = jnp.sum(jnp.where(eye8, adjp2, 0.0))
    link2sq = jnp.sum(adjp * adjp) - 2.0 * tr2 + jnp.sum(sts2 * sts2)
    link2 = jnp.sqrt(jnp.maximum(link2sq, 1e-12)) / 32.0
    reg2 = link2 + ent2

    sts1 = sts_ref[...]
    eye32 = (lax.broadcasted_iota(jnp.int32, (32, 32), 0)
             == lax.broadcasted_iota(jnp.int32, (32, 32), 1))
    tr1 = jnp.sum(jnp.where(eye32, adjp, 0.0))
    link1sq = fro_ref[0, 0] - 2.0 * tr1 + jnp.sum(sts1 * sts1)
    link1 = jnp.sqrt(jnp.maximum(link1sq, 1e-12)) / 10000.0
    ent1 = ent_ref[0, 0] / 10000.0
    reg = (link1 + ent1) * 10.0 + reg2 * 0.1

    acc = bf1_ref[...]
    for r in range(8):
        acc = acc + jnp.dot(x3[r:r + 1, :], wf1_ref[r],
                            preferred_element_type=jnp.float32)
    h1f = jnp.maximum(acc, 0.0)
    out2 = jnp.dot(h1f, wf2_ref[...], preferred_element_type=jnp.float32) \
        + bf2_ref[...]
    outa_ref[...] = jnp.broadcast_to(out2, (8, 128))
    outb_ref[...] = jnp.full((8, 128), reg)


def _tail(adjp, xp, sts, fro, ent, wp2, asp2, adp2, aep2, wf1g, bf1p, wf2p,
          bf2p):
    vm = pl.BlockSpec(memory_space=pltpu.VMEM)
    sm = pl.BlockSpec(memory_space=pltpu.SMEM)
    return pl.pallas_call(
        _tail_body,
        in_specs=[vm, vm, vm, sm, sm, vm, vm, vm, sm, vm, vm, vm, vm],
        out_shape=[
            jax.ShapeDtypeStruct((8, 128), jnp.float32),
            jax.ShapeDtypeStruct((8, 128), jnp.float32),
        ],
    )(adjp, xp, sts, fro, ent, wp2, asp2, adp2, aep2, wf1g, bf1p, wf2p, bf2p)


NW = 32
EC = E_PAD // NW
NCH = EC // 128
NROWS = N_PAD // 16


def _sc_mesh():
    return plsc.VectorSubcoreMesh(core_axis_name="c", subcore_axis_name="s")


def _sc_phase_a(ss, sd, et, k32, src2d, dst2d):
    @functools.partial(
        pl.kernel,
        out_type=[
            jax.ShapeDtypeStruct((2, N_PAD, 32), jnp.float32),
            jax.ShapeDtypeStruct((E_PAD, 32), jnp.float32),
        ],
        mesh=_sc_mesh(),
        compiler_params=pltpu.CompilerParams(use_tc_tiling_on_sc=False),
        scratch_types=[
            pltpu.VMEM((NCH, 128), jnp.int32),
            pltpu.VMEM((NCH, 128), jnp.int32),
            pltpu.VMEM((128, 32), jnp.float32),
            pltpu.VMEM((128, 32), jnp.float32),
            pltpu.VMEM((128, 32), jnp.float32),
            pltpu.VMEM((128, 32), jnp.float32),
            pltpu.VMEM((128, 32), jnp.float32),
            pltpu.VMEM((128, 32), jnp.float32),
            pltpu.VMEM((128, 32), jnp.float32),
            pltpu.VMEM((128, 32), jnp.float32),
            pltpu.VMEM((32,), jnp.float32),
            pltpu.VMEM((NROWS, 32), jnp.float32),
            pltpu.VMEM_SHARED((N_PAD, 32), jnp.float32),
            pltpu.SemaphoreType.DMA,
            pltpu.SemaphoreType.DMA,
            pltpu.SemaphoreType.DMA,
            pltpu.SemaphoreType.DMA,
            pltpu.SemaphoreType.DMA,
            pltpu.SemaphoreType.DMA,
            pltpu.SemaphoreType.DMA,
            pltpu.SemaphoreType.DMA,
        ],
    )
    def k(ss_hbm, sd_hbm, et_hbm, k_hbm, src_hbm, dst_hbm, den_out, ex_out,
          srci, dsti, ssb0, sdb0, etb0, exb0, ssb1, sdb1, etb1, exb1,
          kv, stage, den_sh, gsem0, gsem1, lsem0, lsem1, ssem0, ssem1,
          wsem0, wsem1):
        cid = lax.axis_index("c")
        sid = lax.axis_index("s")
        wid = sid * 2 + cid
        base = wid * EC

        pltpu.sync_copy(k_hbm, kv)
        pltpu.sync_copy(src_hbm.at[pl.ds(wid * NCH, NCH), :], srci)
        pltpu.sync_copy(dst_hbm.at[pl.ds(wid * NCH, NCH), :], dsti)
        k0 = kv[0:16]
        k1 = kv[16:32]

        def zbody(i, _):
            stage[i, 0:16] = jnp.zeros((16,), jnp.float32)
            stage[i, 16:32] = jnp.zeros((16,), jnp.float32)
            return 0

        lax.fori_loop(0, NROWS, zbody, 0)
        pltpu.sync_copy(stage, den_sh.at[pl.ds(sid * NROWS, NROWS)])
        plsc.subcore_barrier()

        def issue_in(r, ssb, sdb, etb, gsem, lsem):
            off = base + r * 128
            pltpu.async_copy(ss_hbm.at[srci.at[r]], ssb, gsem)
            pltpu.async_copy(sd_hbm.at[dsti.at[r]], sdb, gsem)
            pltpu.async_copy(et_hbm.at[pl.ds(off, 128), :], etb, lsem)

        def wait_in(r, ssb, sdb, etb, gsem, lsem):
            off = base + r * 128
            pltpu.make_async_copy(ss_hbm.at[srci.at[r]], ssb, gsem).wait()
            pltpu.make_async_copy(sd_hbm.at[dsti.at[r]], sdb, gsem).wait()
            pltpu.make_async_copy(
                et_hbm.at[pl.ds(off, 128), :], etb, lsem).wait()

        def compute(ssb, sdb, etb, exb):
            def jbody(j, _):
                a0 = ssb[j, 0:16] + sdb[j, 0:16] + etb[j, 0:16]
                a0 = jnp.maximum(a0, 0.2 * a0) - k0
                exb[j, 0:16] = jnp.exp(a0)
                a1 = ssb[j, 16:32] + sdb[j, 16:32] + etb[j, 16:32]
                a1 = jnp.maximum(a1, 0.2 * a1) - k1
                exb[j, 16:32] = jnp.exp(a1)
                return 0

            lax.fori_loop(0, 128, jbody, 0, unroll=4)

        def issue_out(r, exb, ssem, wsem):
            off = base + r * 128
            pltpu.async_copy(exb, den_sh.at[dsti.at[r]], ssem, add=True)
            pltpu.async_copy(exb, ex_out.at[pl.ds(off, 128), :], wsem)

        def wait_out(r, exb, ssem, wsem):
            off = base + r * 128
            pltpu.make_async_copy(
                exb, den_sh.at[dsti.at[r]], ssem).wait()
            pltpu.make_async_copy(
                exb, ex_out.at[pl.ds(off, 128), :], wsem).wait()

        issue_in(0, ssb0, sdb0, etb0, gsem0, lsem0)
        issue_in(1, ssb1, sdb1, etb1, gsem1, lsem1)

        def half(c, r, ssb, sdb, etb, exb, gsem, lsem, ssem, wsem):
            wait_in(r, ssb, sdb, etb, gsem, lsem)

            @pl.when(c > 0)
            def _():
                wait_out(r - 2, exb, ssem, wsem)

            compute(ssb, sdb, etb, exb)
            issue_out(r, exb, ssem, wsem)

            @pl.when(c < NCH // 2 - 1)
            def _():
                issue_in(r + 2, ssb, sdb, etb, gsem, lsem)

        def chunk2(c, _):
            half(c, 2 * c, ssb0, sdb0, etb0, exb0, gsem0, lsem0, ssem0,
                 wsem0)
            half(c, 2 * c + 1, ssb1, sdb1, etb1, exb1, gsem1, lsem1, ssem1,
                 wsem1)
            return 0

        lax.fori_loop(0, NCH // 2, chunk2, 0)
        wait_out(NCH - 2, exb0, ssem0, wsem0)
        wait_out(NCH - 1, exb1, ssem1, wsem1)
        plsc.subcore_barrier()
        pltpu.sync_copy(
            den_sh.at[pl.ds(sid * NROWS, NROWS)],
            den_out.at[cid, pl.ds(sid * NROWS, NROWS), :])

    return k(ss, sd, et, k32, src2d, dst2d)


def _sc_phase_b(ex, rden, h, src2d, dst2d):
    @functools.partial(
        pl.kernel,
        out_type=jax.ShapeDtypeStruct((2, N_PAD, 32), jnp.float32),
        mesh=_sc_mesh(),
        compiler_params=pltpu.CompilerParams(use_tc_tiling_on_sc=False),
        scratch_types=[
            pltpu.VMEM((NCH, 128), jnp.int32),
            pltpu.VMEM((NCH, 128), jnp.int32),
            pltpu.VMEM((128, 32), jnp.float32),
            pltpu.VMEM((128, 32), jnp.float32),
            pltpu.VMEM((128, 32), jnp.float32),
            pltpu.VMEM((128, 32), jnp.float32),
            pltpu.VMEM((128, 32), jnp.float32),
            pltpu.VMEM((128, 32), jnp.float32),
            pltpu.VMEM((128, 32), jnp.float32),
            pltpu.VMEM((128, 32), jnp.float32),
            pltpu.VMEM((NROWS, 32), jnp.float32),
            pltpu.VMEM_SHARED((N_PAD, 32), jnp.float32),
            pltpu.SemaphoreType.DMA,
            pltpu.SemaphoreType.DMA,
            pltpu.SemaphoreType.DMA,
            pltpu.SemaphoreType.DMA,
            pltpu.SemaphoreType.DMA,
            pltpu.SemaphoreType.DMA,
        ],
    )
    def k(ex_hbm, rd_hbm, h_hbm, src_hbm, dst_hbm, agg_out,
          srci, dsti, exb0, rdb0, hb0, msgb0, exb1, rdb1, hb1, msgb1,
          stage, agg_sh, gsem0, gsem1, lsem0, lsem1, ssem0, ssem1):
        cid = lax.axis_index("c")
        sid = lax.axis_index("s")
        wid = sid * 2 + cid
        base = wid * EC

        pltpu.sync_copy(src_hbm.at[pl.ds(wid * NCH, NCH), :], srci)
        pltpu.sync_copy(dst_hbm.at[pl.ds(wid * NCH, NCH), :], dsti)

        def zbody(i, _):
            stage[i, 0:16] = jnp.zeros((16,), jnp.float32)
            stage[i, 16:32] = jnp.zeros((16,), jnp.float32)
            return 0

        lax.fori_loop(0, NROWS, zbody, 0)
        pltpu.sync_copy(stage, agg_sh.at[pl.ds(sid * NROWS, NROWS)])
        plsc.subcore_barrier()

        def issue_in(r, rdb, hb, exb, gsem, lsem):
            off = base + r * 128
            pltpu.async_copy(rd_hbm.at[dsti.at[r]], rdb, gsem)
            pltpu.async_copy(h_hbm.at[srci.at[r]], hb, gsem)
            pltpu.async_copy(ex_hbm.at[pl.ds(off, 128), :], exb, lsem)

        def wait_in(r, rdb, hb, exb, gsem, lsem):
            off = base + r * 128
            pltpu.make_async_copy(rd_hbm.at[dsti.at[r]], rdb, gsem).wait()
            pltpu.make_async_copy(h_hbm.at[srci.at[r]], hb, gsem).wait()
            pltpu.make_async_copy(
                ex_hbm.at[pl.ds(off, 128), :], exb, lsem).wait()

        def compute(rdb, hb, exb, msgb):
            def jbody(j, _):
                msgb[j, 0:16] = hb[j, 0:16] * exb[j, 0:16] * rdb[j, 0:16]
                msgb[j, 16:32] = hb[j, 16:32] * exb[j, 16:32] * rdb[j, 16:32]
                return 0

            lax.fori_loop(0, 128, jbody, 0, unroll=4)

        issue_in(0, rdb0, hb0, exb0, gsem0, lsem0)
        issue_in(1, rdb1, hb1, exb1, gsem1, lsem1)

        def half(c, r, rdb, hb, exb, msgb, gsem, lsem, ssem):
            wait_in(r, rdb, hb, exb, gsem, lsem)

            @pl.when(c > 0)
            def _():
                pltpu.make_async_copy(
                    msgb, agg_sh.at[dsti.at[r - 2]], ssem).wait()

            compute(rdb, hb, exb, msgb)
            pltpu.async_copy(msgb, agg_sh.at[dsti.at[r]], ssem, add=True)

            @pl.when(c < NCH // 2 - 1)
            def _():
                issue_in(r + 2, rdb, hb, exb, gsem, lsem)

        def chunk2(c, _):
            half(c, 2 * c, rdb0, hb0, exb0, msgb0, gsem0, lsem0, ssem0)
            half(c, 2 * c + 1, rdb1, hb1, exb1, msgb1, gsem1, lsem1, ssem1)
            return 0

        lax.fori_loop(0, NCH // 2, chunk2, 0)
        pltpu.make_async_copy(
            msgb0, agg_sh.at[dsti.at[NCH - 2]], ssem0).wait()
        pltpu.make_async_copy(
            msgb1, agg_sh.at[dsti.at[NCH - 1]], ssem1).wait()
        plsc.subcore_barrier()
        pltpu.sync_copy(
            agg_sh.at[pl.ds(sid * NROWS, NROWS)],
            agg_out.at[cid, pl.ds(sid * NROWS, NROWS), :])

    return k(ex, rden, h, src2d, dst2d)


def _conv_segops(ss, sd, et, k32, h, srcp, dstp):
    dparts, ex = _sc_phase_a(ss, sd, et, k32, srcp, dstp)
    rden = _rden(dparts)
    return _sc_phase_b(ex, rden, h, srcp, dstp)


def kernel(x, edge_index, edge_attr, y, adj, W1, a_src1, a_dst1, a_edge1,
           Wp1, a_src_p1, a_dst_p1, a_edge_p1, Wp2, a_src_p2, a_dst_p2,
           a_edge_p2, Wf1, bf1, Wf2, bf2):
    n, dfeat = x.shape
    e = edge_index.shape[1]

    xp_in = jnp.pad(x, ((0, N_PAD - n), (0, 0)))
    srcp = jnp.concatenate(
        [edge_index[0].astype(jnp.int32),
         jnp.zeros((E_PAD - e,), jnp.int32)]).reshape(E_PAD // 128, 128)
    dstp = jnp.concatenate(
        [edge_index[1].astype(jnp.int32),
         jnp.full((E_PAD - e,), n, jnp.int32)]).reshape(E_PAD // 128, 128)
    eap = jnp.pad(edge_attr, ((0, E_PAD - e), (0, 0)))

    w1p = jnp.pad(W1, ((0, 0), (0, 2)))
    asrc1 = jnp.zeros((32, 32), jnp.float32)
    adst1 = jnp.zeros((32, 32), jnp.float32)
    ae1p = jnp.zeros((4, 32), jnp.float32)
    for hh in range(5):
        blk_s = jnp.broadcast_to(a_src1[hh][:, None], (6, 6))
        blk_d = jnp.broadcast_to(a_dst1[hh][:, None], (6, 6))
        asrc1 = asrc1.at[hh * 6:(hh + 1) * 6, hh * 6:(hh + 1) * 6].set(blk_s)
        adst1 = adst1.at[hh * 6:(hh + 1) * 6, hh * 6:(hh + 1) * 6].set(blk_d)
        ae1p = ae1p.at[:, hh * 6:(hh + 1) * 6].set(
            jnp.broadcast_to(a_edge1[:, hh:hh + 1], (4, 6)))

    wp1p = jnp.pad(Wp1, ((0, 2), (0, 0)))
    asrc2 = jnp.broadcast_to(a_src_p1[0][:, None], (32, 32))
    adst2 = jnp.broadcast_to(a_dst_p1[0][:, None], (32, 32))
    ae2p = jnp.broadcast_to(a_edge_p1, (4, 32))

    h1p, ss1, sd1 = _node_prep(xp_in, w1p, asrc1, adst1)
    et1, et2 = _edge_prep(eap, ae1p, ae2p)
    k1 = _maxes(ss1, sd1, et1)[0]

    aggparts1 = _conv_segops(ss1, sd1, et1, k1, h1p, srcp, dstp)

    x1, h2p, ss2, sd2 = _prep2(aggparts1, wp1p, asrc2, adst2)
    k2 = _maxes(ss2, sd2, et2)[0]

    aggparts2 = _conv_segops(ss2, sd2, et2, k2, h2p, srcp, dstp)

    s, ent_sum = _smax(aggparts2, n)
    adjp, xp, sts, fro = _adj_pass(adj, s, x1)

    wp2p = jnp.pad(Wp2, ((0, 2), (0, 4)))
    asp2 = jnp.zeros((8, 8), jnp.float32).at[0:4, 0].set(a_src_p2[0])
    adp2 = jnp.zeros((8, 8), jnp.float32).at[0:4, 0].set(a_dst_p2[0])
    aep2 = a_edge_p2.reshape(1, 1)
    wf1g = jnp.zeros((8, 32, 32), jnp.float32).at[0:4, 0:30, :].set(
        Wf1.reshape(4, 30, 32))
    bf1p = bf1.reshape(1, 32)
    wf2p = jnp.zeros((32, 128), jnp.float32).at[:, 0:2].set(Wf2)
    bf2p = jnp.zeros((1, 128), jnp.float32).at[0, 0:2].set(bf2)

    outa, outb = _tail(adjp, xp, sts, fro, ent_sum, wp2p, asp2, adp2, aep2,
                       wf1g, bf1p, wf2p, bf2p)
    return outa[0:1, 0:2], outb[0, 0]


# speedup vs baseline: 21.2671x; 1.1125x over previous
import functools

import jax
import jax.numpy as jnp
from jax import lax
from jax.experimental import pallas as pl
from jax.experimental.pallas import tpu as pltpu
from jax.experimental.pallas import tpu_sc as plsc

N_PAD = 10240
E_PAD = 163840
BN = 1024
BE = 2048
ADJ_BR = 1000
ADJ_BC = 2048


def _node_prep_body(x_ref, w_ref, asrc_ref, adst_ref, h_ref, ss_ref, sd_ref,
                    m_ref):
    i = pl.program_id(0)
    h = jnp.dot(x_ref[...], w_ref[...], preferred_element_type=jnp.float32)
    h_ref[...] = h
    ss = jnp.dot(h, asrc_ref[...], preferred_element_type=jnp.float32)
    sd = jnp.dot(h, adst_ref[...], preferred_element_type=jnp.float32)
    ss_ref[...] = ss
    sd_ref[...] = sd

    @pl.when(i == 0)
    def _():
        m_ref[...] = jnp.full_like(m_ref, -1e30)

    cm = jnp.max(ss, axis=0) + jnp.max(sd, axis=0)
    m_ref[...] = jnp.maximum(m_ref[...],
                             jnp.broadcast_to(cm[None, :], (8, 32)))


def _node_prep(xp, w, asrc, adst):
    npad, din = xp.shape
    dh = w.shape[1]
    grid = npad // BN
    return pl.pallas_call(
        _node_prep_body,
        grid=(grid,),
        in_specs=[
            pl.BlockSpec((BN, din), lambda i: (i, 0)),
            pl.BlockSpec((din, dh), lambda i: (0, 0)),
            pl.BlockSpec((dh, 32), lambda i: (0, 0)),
            pl.BlockSpec((dh, 32), lambda i: (0, 0)),
        ],
        out_specs=[
            pl.BlockSpec((BN, dh), lambda i: (i, 0)),
            pl.BlockSpec((BN, 32), lambda i: (i, 0)),
            pl.BlockSpec((BN, 32), lambda i: (i, 0)),
            pl.BlockSpec((8, 32), lambda i: (0, 0)),
        ],
        out_shape=[
            jax.ShapeDtypeStruct((npad, dh), jnp.float32),
            jax.ShapeDtypeStruct((npad, 32), jnp.float32),
            jax.ShapeDtypeStruct((npad, 32), jnp.float32),
            jax.ShapeDtypeStruct((8, 32), jnp.float32),
        ],
    )(xp, w, asrc, adst)


def _edge_prep_body(ea_ref, ae1_ref, ae2_ref, et1_ref, et2_ref, m1_ref,
                    m2_ref):
    i = pl.program_id(0)
    ea = ea_ref[...]
    et1 = jnp.dot(ea, ae1_ref[...], preferred_element_type=jnp.float32)
    et2 = jnp.dot(ea, ae2_ref[...], preferred_element_type=jnp.float32)
    et1_ref[...] = et1
    et2_ref[...] = et2

    @pl.when(i == 0)
    def _():
        m1_ref[...] = jnp.full_like(m1_ref, -1e30)
        m2_ref[...] = jnp.full_like(m2_ref, -1e30)

    cm1 = jnp.max(et1, axis=0)
    cm2 = jnp.max(et2, axis=0)
    m1_ref[...] = jnp.maximum(m1_ref[...],
                              jnp.broadcast_to(cm1[None, :], (8, 32)))
    m2_ref[...] = jnp.maximum(m2_ref[...],
                              jnp.broadcast_to(cm2[None, :], (8, 32)))


def _edge_prep(eap, ae1, ae2):
    epad, de = eap.shape
    grid = epad // BE
    return pl.pallas_call(
        _edge_prep_body,
        grid=(grid,),
        in_specs=[
            pl.BlockSpec((BE, de), lambda i: (i, 0)),
            pl.BlockSpec((de, 32), lambda i: (0, 0)),
            pl.BlockSpec((de, 32), lambda i: (0, 0)),
        ],
        out_specs=[
            pl.BlockSpec((BE, 32), lambda i: (i, 0)),
            pl.BlockSpec((BE, 32), lambda i: (i, 0)),
            pl.BlockSpec((8, 32), lambda i: (0, 0)),
            pl.BlockSpec((8, 32), lambda i: (0, 0)),
        ],
        out_shape=[
            jax.ShapeDtypeStruct((epad, 32), jnp.float32),
            jax.ShapeDtypeStruct((epad, 32), jnp.float32),
            jax.ShapeDtypeStruct((8, 32), jnp.float32),
            jax.ShapeDtypeStruct((8, 32), jnp.float32),
        ],
    )(eap, ae1, ae2)


def _rden_body(d_ref, r_ref):
    d = d_ref[0] + d_ref[1]
    r_ref[...] = 1.0 / (d + 1e-16)


def _rden(dparts):
    npad = dparts.shape[1]
    return pl.pallas_call(
        _rden_body,
        out_shape=jax.ShapeDtypeStruct((npad, 32), jnp.float32),
    )(dparts)


def _prep2_body(a_ref, w_ref, asrc_ref, adst_ref, x1_ref, h2_ref, ss_ref,
                sd_ref, m_ref):
    i = pl.program_id(0)
    x1 = a_ref[0] + a_ref[1]
    x1_ref[...] = x1
    h2 = jnp.dot(x1, w_ref[...], preferred_element_type=jnp.float32)
    h2_ref[...] = h2
    ss = jnp.dot(h2, asrc_ref[...], preferred_element_type=jnp.float32)
    sd = jnp.dot(h2, adst_ref[...], preferred_element_type=jnp.float32)
    ss_ref[...] = ss
    sd_ref[...] = sd

    @pl.when(i == 0)
    def _():
        m_ref[...] = jnp.full_like(m_ref, -1e30)

    cm = jnp.max(ss, axis=0) + jnp.max(sd, axis=0)
    m_ref[...] = jnp.maximum(m_ref[...],
                             jnp.broadcast_to(cm[None, :], (8, 32)))


def _prep2(aggparts, wp, asrc, adst):
    npad = aggparts.shape[1]
    grid = npad // BN
    return pl.pallas_call(
        _prep2_body,
        grid=(grid,),
        in_specs=[
            pl.BlockSpec((2, BN, 32), lambda i: (0, i, 0)),
            pl.BlockSpec((32, 32), lambda i: (0, 0)),
            pl.BlockSpec((32, 32), lambda i: (0, 0)),
            pl.BlockSpec((32, 32), lambda i: (0, 0)),
        ],
        out_specs=[
            pl.BlockSpec((BN, 32), lambda i: (i, 0)),
            pl.BlockSpec((BN, 32), lambda i: (i, 0)),
            pl.BlockSpec((BN, 32), lambda i: (i, 0)),
            pl.BlockSpec((BN, 32), lambda i: (i, 0)),
            pl.BlockSpec((8, 32), lambda i: (0, 0)),
        ],
        out_shape=[
            jax.ShapeDtypeStruct((npad, 32), jnp.float32),
            jax.ShapeDtypeStruct((npad, 32), jnp.float32),
            jax.ShapeDtypeStruct((npad, 32), jnp.float32),
            jax.ShapeDtypeStruct((npad, 32), jnp.float32),
            jax.ShapeDtypeStruct((8, 32), jnp.float32),
        ],
    )(aggparts, wp, asrc, adst)


def _smax_body(s1_ref, s_ref, ent_ref, *, nreal):
    i = pl.program_id(0)
    z = s1_ref[0] + s1_ref[1]
    m = jnp.max(z, axis=1, keepdims=True)
    e = jnp.exp(z - m)
    sm = e / jnp.sum(e, axis=1, keepdims=True)
    rid = i * BN + lax.broadcasted_iota(jnp.int32, sm.shape, 0)
    sm = jnp.where(rid < nreal, sm, 0.0)
    s_ref[...] = sm
    ent = -jnp.sum(sm * jnp.log(sm + 1e-15))

    @pl.when(i == 0)
    def _():
        ent_ref[0, 0] = 0.0

    ent_ref[0, 0] += ent


def _smax(s1parts, nreal):
    npad = s1parts.shape[1]
    grid = npad // BN
    return pl.pallas_call(
        functools.partial(_smax_body, nreal=nreal),
        grid=(grid,),
        in_specs=[pl.BlockSpec((2, BN, 32), lambda i: (0, i, 0))],
        out_specs=[
            pl.BlockSpec((BN, 32), lambda i: (i, 0)),
            pl.BlockSpec((1, 1), lambda i: (0, 0),
                         memory_space=pltpu.SMEM),
        ],
        out_shape=[
            jax.ShapeDtypeStruct((npad, 32), jnp.float32),
            jax.ShapeDtypeStruct((1, 1), jnp.float32),
        ],
    )(s1parts)


def _adj_body(adj_ref, sk_ref, si_ref, x1_ref, adjp_ref, xp_ref, sts_ref,
              fro_ref, tmp_ref, *, nk, ncols):
    i = pl.program_id(0)
    k = pl.program_id(1)

    blk = adj_ref[...]
    colid = k * ADJ_BC + lax.broadcasted_iota(jnp.int32, blk.shape, 1)
    blk = jnp.where(colid < ncols, blk, 0.0)

    @pl.when(jnp.logical_and(i == 0, k == 0))
    def _():
        adjp_ref[...] = jnp.zeros_like(adjp_ref)
        xp_ref[...] = jnp.zeros_like(xp_ref)
        sts_ref[...] = jnp.zeros_like(sts_ref)
        fro_ref[0, 0] = 0.0

    fro_ref[0, 0] += jnp.sum(blk * blk)

    part = jnp.dot(blk, sk_ref[...], preferred_element_type=jnp.float32)

    @pl.when(k == 0)
    def _():
        tmp_ref[...] = part
        si = si_ref[...]
        xp_ref[...] += lax.dot_general(
            si, x1_ref[...], (((0,), (0,)), ((), ())),
            preferred_element_type=jnp.float32)
        sts_ref[...] += lax.dot_general(
            si, si, (((0,), (0,)), ((), ())),
            preferred_element_type=jnp.float32)

    @pl.when(k > 0)
    def _():
        tmp_ref[...] += part

    @pl.when(k == nk - 1)
    def _():
        adjp_ref[...] += lax.dot_general(
            si_ref[...], tmp_ref[...], (((0,), (0,)), ((), ())),
            preferred_element_type=jnp.float32)


def _adj_pass(adj, s, x1):
    nrows, ncols = adj.shape
    ni = nrows // ADJ_BR
    nk = (ncols + ADJ_BC - 1) // ADJ_BC
    return pl.pallas_call(
        functools.partial(_adj_body, nk=nk, ncols=ncols),
        grid=(ni, nk),
        in_specs=[
            pl.BlockSpec((ADJ_BR, ADJ_BC), lambda i, k: (i, k)),
            pl.BlockSpec((ADJ_BC, 32), lambda i, k: (k, 0)),
            pl.BlockSpec((ADJ_BR, 32), lambda i, k: (i, 0)),
            pl.BlockSpec((ADJ_BR, 32), lambda i, k: (i, 0)),
        ],
        out_specs=[
            pl.BlockSpec((32, 32), lambda i, k: (0, 0)),
            pl.BlockSpec((32, 32), lambda i, k: (0, 0)),
            pl.BlockSpec((32, 32), lambda i, k: (0, 0)),
            pl.BlockSpec((1, 1), lambda i, k: (0, 0),
                         memory_space=pltpu.SMEM),
        ],
        out_shape=[
            jax.ShapeDtypeStruct((32, 32), jnp.float32),
            jax.ShapeDtypeStruct((32, 32), jnp.float32),
            jax.ShapeDtypeStruct((32, 32), jnp.float32),
            jax.ShapeDtypeStruct((1, 1), jnp.float32),
        ],
        scratch_shapes=[pltpu.VMEM((ADJ_BR, 32), jnp.float32)],
        compiler_params=pltpu.CompilerParams(
            dimension_semantics=("arbitrary", "arbitrary")),
    )(adj, s, s, x1)


def _tail_body(adjp_ref, xp_ref, sts_ref, fro_ref, ent_ref, wp2_ref, asp2_ref,
               adp2_ref, aep2_ref, wf1_ref, bf1_ref, wf2_ref, bf2_ref,
               outa_ref, outb_ref):
    adjp = adjp_ref[...]
    x2 = xp_ref[...]
    h3 = jnp.dot(x2, wp2_ref[...], preferred_element_type=jnp.float32)
    ss3 = jnp.dot(h3, asp2_ref[...], preferred_element_type=jnp.float32)
    sd3m = lax.dot_general(adp2_ref[...], h3, (((0,), (1,)), ((), ())),
                           preferred_element_type=jnp.float32)
    alpha = ss3[:, 0:1] + sd3m[0:1, :] + adjp * aep2_ref[0, 0]
    alpha = jnp.maximum(alpha, 0.2 * alpha)
    cmax = jnp.max(alpha, axis=0, keepdims=True)
    ex = jnp.exp(alpha - cmax)
    att = ex / (jnp.sum(ex, axis=0, keepdims=True) + 1e-16)
    s2 = lax.dot_general(att, h3, (((0,), (0,)), ((), ())),
                         preferred_element_type=jnp.float32)

    colmask = lax.broadcasted_iota(jnp.int32, s2.shape, 1) < 4
    z = jnp.where(colmask, s2, -1e30)
    m2 = jnp.max(z, axis=1, keepdims=True)
    e2 = jnp.where(colmask, jnp.exp(z - m2), 0.0)
    s2s = e2 / jnp.sum(e2, axis=1, keepdims=True)
    ent2 = -jnp.sum(s2s * jnp.log(s2s + 1e-15)) / 32.0

    x3 = lax.dot_general(s2s, x2, (((0,), (0,)), ((), ())),
                         preferred_element_type=jnp.float32)
    adjs2 = jnp.dot(adjp, s2s, preferred_element_type=jnp.float32)
    adjp2 = lax.dot_general(s2s, adjs2, (((0,), (0,)), ((), ())),
                            preferred_element_type=jnp.float32)
    sts2 = lax.dot_general(s2s, s2s, (((0,), (0,)), ((), ())),
                           preferred_element_type=jnp.float32)
    eye8 = (lax.broadcasted_iota(jnp.int32, (8, 8), 0)
            == lax.broadcasted_iota(jnp.int32, (8, 8), 1))
    tr2 = jnp.sum(jnp.where(eye8, adjp2, 0.0))
    link2sq = jnp.sum(adjp * adjp) - 2.0 * tr2 + jnp.sum(sts2 * sts2)
    link2 = jnp.sqrt(jnp.maximum(link2sq, 1e-12)) / 32.0
    reg2 = link2 + ent2

    sts1 = sts_ref[...]
    eye32 = (lax.broadcasted_iota(jnp.int32, (32, 32), 0)
             == lax.broadcasted_iota(jnp.int32, (32, 32), 1))
    tr1 = jnp.sum(jnp.where(eye32, adjp, 0.0))
    link1sq = fro_ref[0, 0] - 2.0 * tr1 + jnp.sum(sts1 * sts1)
    link1 = jnp.sqrt(jnp.maximum(link1sq, 1e-12)) / 10000.0
    ent1 = ent_ref[0, 0] / 10000.0
    reg = (link1 + ent1) * 10.0 + reg2 * 0.1

    acc = bf1_ref[...]
    for r in range(8):
        acc = acc + jnp.dot(x3[r:r + 1, :], wf1_ref[r],
                            preferred_element_type=jnp.float32)
    h1f = jnp.maximum(acc, 0.0)
    out2 = jnp.dot(h1f, wf2_ref[...], preferred_element_type=jnp.float32) \
        + bf2_ref[...]
    outa_ref[...] = jnp.broadcast_to(out2, (8, 128))
    outb_ref[...] = jnp.full((8, 128), reg)


def _tail(adjp, xp, sts, fro, ent, wp2, asp2, adp2, aep2, wf1g, bf1p, wf2p,
          bf2p):
    vm = pl.BlockSpec(memory_space=pltpu.VMEM)
    sm = pl.BlockSpec(memory_space=pltpu.SMEM)
    return pl.pallas_call(
        _tail_body,
        in_specs=[vm, vm, vm, sm, sm, vm, vm, vm, sm, vm, vm, vm, vm],
        out_shape=[
            jax.ShapeDtypeStruct((8, 128), jnp.float32),
            jax.ShapeDtypeStruct((8, 128), jnp.float32),
        ],
    )(adjp, xp, sts, fro, ent, wp2, asp2, adp2, aep2, wf1g, bf1p, wf2p, bf2p)


NW = 32
EC = E_PAD // NW
NCH = EC // 128
NROWS = N_PAD // 16


def _sc_mesh():
    return plsc.VectorSubcoreMesh(core_axis_name="c", subcore_axis_name="s")


def _sc_phase_a(ss, sd, et, k32, src2d, dst2d):
    @functools.partial(
        pl.kernel,
        out_type=[
            jax.ShapeDtypeStruct((2, N_PAD, 32), jnp.float32),
            jax.ShapeDtypeStruct((E_PAD, 32), jnp.float32),
        ],
        mesh=_sc_mesh(),
        compiler_params=pltpu.CompilerParams(use_tc_tiling_on_sc=False),
        scratch_types=[
            pltpu.VMEM((NCH, 128), jnp.int32),
            pltpu.VMEM((NCH, 128), jnp.int32),
            pltpu.VMEM((128, 32), jnp.float32),
            pltpu.VMEM((128, 32), jnp.float32),
            pltpu.VMEM((128, 32), jnp.float32),
            pltpu.VMEM((128, 32), jnp.float32),
            pltpu.VMEM((128, 32), jnp.float32),
            pltpu.VMEM((128, 32), jnp.float32),
            pltpu.VMEM((128, 32), jnp.float32),
            pltpu.VMEM((128, 32), jnp.float32),
            pltpu.VMEM((32,), jnp.float32),
            pltpu.VMEM((NROWS, 32), jnp.float32),
            pltpu.VMEM_SHARED((N_PAD, 32), jnp.float32),
            pltpu.SemaphoreType.DMA,
            pltpu.SemaphoreType.DMA,
            pltpu.SemaphoreType.DMA,
            pltpu.SemaphoreType.DMA,
            pltpu.SemaphoreType.DMA,
            pltpu.SemaphoreType.DMA,
            pltpu.SemaphoreType.DMA,
            pltpu.SemaphoreType.DMA,
        ],
    )
    def k(ss_hbm, sd_hbm, et_hbm, k_hbm, src_hbm, dst_hbm, den_out, ex_out,
          srci, dsti, ssb0, sdb0, etb0, exb0, ssb1, sdb1, etb1, exb1,
          kv, stage, den_sh, gsem0, gsem1, lsem0, lsem1, ssem0, ssem1,
          wsem0, wsem1):
        cid = lax.axis_index("c")
        sid = lax.axis_index("s")
        wid = sid * 2 + cid
        base = wid * EC

        pltpu.sync_copy(k_hbm, kv)
        pltpu.sync_copy(src_hbm.at[pl.ds(wid * NCH, NCH), :], srci)
        pltpu.sync_copy(dst_hbm.at[pl.ds(wid * NCH, NCH), :], dsti)
        k0 = kv[0:16]
        k1 = kv[16:32]

        def zbody(i, _):
            stage[i, 0:16] = jnp.zeros((16,), jnp.float32)
            stage[i, 16:32] = jnp.zeros((16,), jnp.float32)
            return 0

        lax.fori_loop(0, NROWS, zbody, 0)
        pltpu.sync_copy(stage, den_sh.at[pl.ds(sid * NROWS, NROWS)])
        plsc.subcore_barrier()

        def issue_in(r, ssb, sdb, etb, gsem, lsem):
            off = base + r * 128
            pltpu.async_copy(ss_hbm.at[srci.at[r]], ssb, gsem)
            pltpu.async_copy(sd_hbm.at[dsti.at[r]], sdb, gsem)
            pltpu.async_copy(et_hbm.at[pl.ds(off, 128), :], etb, lsem)

        def wait_in(r, ssb, sdb, etb, gsem, lsem):
            off = base + r * 128
            pltpu.make_async_copy(ss_hbm.at[srci.at[r]], ssb, gsem).wait()
            pltpu.make_async_copy(sd_hbm.at[dsti.at[r]], sdb, gsem).wait()
            pltpu.make_async_copy(
                et_hbm.at[pl.ds(off, 128), :], etb, lsem).wait()

        def compute(ssb, sdb, etb, exb):
            def jbody(j, _):
                a0 = ssb[j, 0:16] + sdb[j, 0:16] + etb[j, 0:16]
                a0 = jnp.maximum(a0, 0.2 * a0) - k0
                exb[j, 0:16] = jnp.exp(a0)
                a1 = ssb[j, 16:32] + sdb[j, 16:32] + etb[j, 16:32]
                a1 = jnp.maximum(a1, 0.2 * a1) - k1
                exb[j, 16:32] = jnp.exp(a1)
                return 0

            lax.fori_loop(0, 128, jbody, 0, unroll=4)

        def issue_out(r, exb, ssem, wsem):
            off = base + r * 128
            pltpu.async_copy(exb, den_sh.at[dsti.at[r]], ssem, add=True)
            pltpu.async_copy(exb, ex_out.at[pl.ds(off, 128), :], wsem)

        def wait_out(r, exb, ssem, wsem):
            off = base + r * 128
            pltpu.make_async_copy(
                exb, den_sh.at[dsti.at[r]], ssem).wait()
            pltpu.make_async_copy(
                exb, ex_out.at[pl.ds(off, 128), :], wsem).wait()

        issue_in(0, ssb0, sdb0, etb0, gsem0, lsem0)
        issue_in(1, ssb1, sdb1, etb1, gsem1, lsem1)

        def half(c, r, ssb, sdb, etb, exb, gsem, lsem, ssem, wsem):
            wait_in(r, ssb, sdb, etb, gsem, lsem)

            @pl.when(c > 0)
            def _():
                wait_out(r - 2, exb, ssem, wsem)

            compute(ssb, sdb, etb, exb)
            issue_out(r, exb, ssem, wsem)

            @pl.when(c < NCH // 2 - 1)
            def _():
                issue_in(r + 2, ssb, sdb, etb, gsem, lsem)

        def chunk2(c, _):
            half(c, 2 * c, ssb0, sdb0, etb0, exb0, gsem0, lsem0, ssem0,
                 wsem0)
            half(c, 2 * c + 1, ssb1, sdb1, etb1, exb1, gsem1, lsem1, ssem1,
                 wsem1)
            return 0

        lax.fori_loop(0, NCH // 2, chunk2, 0)
        wait_out(NCH - 2, exb0, ssem0, wsem0)
        wait_out(NCH - 1, exb1, ssem1, wsem1)
        plsc.subcore_barrier()
        pltpu.sync_copy(
            den_sh.at[pl.ds(sid * NROWS, NROWS)],
            den_out.at[cid, pl.ds(sid * NROWS, NROWS), :])

    return k(ss, sd, et, k32, src2d, dst2d)


def _sc_phase_b(ex, rden, h, src2d, dst2d):
    @functools.partial(
        pl.kernel,
        out_type=jax.ShapeDtypeStruct((2, N_PAD, 32), jnp.float32),
        mesh=_sc_mesh(),
        compiler_params=pltpu.CompilerParams(use_tc_tiling_on_sc=False),
        scratch_types=[
            pltpu.VMEM((NCH, 128), jnp.int32),
            pltpu.VMEM((NCH, 128), jnp.int32),
            pltpu.VMEM((128, 32), jnp.float32),
            pltpu.VMEM((128, 32), jnp.float32),
            pltpu.VMEM((128, 32), jnp.float32),
            pltpu.VMEM((128, 32), jnp.float32),
            pltpu.VMEM((128, 32), jnp.float32),
            pltpu.VMEM((128, 32), jnp.float32),
            pltpu.VMEM((128, 32), jnp.float32),
            pltpu.VMEM((128, 32), jnp.float32),
            pltpu.VMEM((NROWS, 32), jnp.float32),
            pltpu.VMEM_SHARED((N_PAD, 32), jnp.float32),
            pltpu.SemaphoreType.DMA,
            pltpu.SemaphoreType.DMA,
            pltpu.SemaphoreType.DMA,
            pltpu.SemaphoreType.DMA,
            pltpu.SemaphoreType.DMA,
            pltpu.SemaphoreType.DMA,
        ],
    )
    def k(ex_hbm, rd_hbm, h_hbm, src_hbm, dst_hbm, agg_out,
          srci, dsti, exb0, rdb0, hb0, msgb0, exb1, rdb1, hb1, msgb1,
          stage, agg_sh, gsem0, gsem1, lsem0, lsem1, ssem0, ssem1):
        cid = lax.axis_index("c")
        sid = lax.axis_index("s")
        wid = sid * 2 + cid
        base = wid * EC

        pltpu.sync_copy(src_hbm.at[pl.ds(wid * NCH, NCH), :], srci)
        pltpu.sync_copy(dst_hbm.at[pl.ds(wid * NCH, NCH), :], dsti)

        def zbody(i, _):
            stage[i, 0:16] = jnp.zeros((16,), jnp.float32)
            stage[i, 16:32] = jnp.zeros((16,), jnp.float32)
            return 0

        lax.fori_loop(0, NROWS, zbody, 0)
        pltpu.sync_copy(stage, agg_sh.at[pl.ds(sid * NROWS, NROWS)])
        plsc.subcore_barrier()

        def issue_in(r, rdb, hb, exb, gsem, lsem):
            off = base + r * 128
            pltpu.async_copy(rd_hbm.at[dsti.at[r]], rdb, gsem)
            pltpu.async_copy(h_hbm.at[srci.at[r]], hb, gsem)
            pltpu.async_copy(ex_hbm.at[pl.ds(off, 128), :], exb, lsem)

        def wait_in(r, rdb, hb, exb, gsem, lsem):
            off = base + r * 128
            pltpu.make_async_copy(rd_hbm.at[dsti.at[r]], rdb, gsem).wait()
            pltpu.make_async_copy(h_hbm.at[srci.at[r]], hb, gsem).wait()
            pltpu.make_async_copy(
                ex_hbm.at[pl.ds(off, 128), :], exb, lsem).wait()

        def compute(rdb, hb, exb, msgb):
            def jbody(j, _):
                msgb[j, 0:16] = hb[j, 0:16] * exb[j, 0:16] * rdb[j, 0:16]
                msgb[j, 16:32] = hb[j, 16:32] * exb[j, 16:32] * rdb[j, 16:32]
                return 0

            lax.fori_loop(0, 128, jbody, 0, unroll=4)

        issue_in(0, rdb0, hb0, exb0, gsem0, lsem0)
        issue_in(1, rdb1, hb1, exb1, gsem1, lsem1)

        def half(c, r, rdb, hb, exb, msgb, gsem, lsem, ssem):
            wait_in(r, rdb, hb, exb, gsem, lsem)

            @pl.when(c > 0)
            def _():
                pltpu.make_async_copy(
                    msgb, agg_sh.at[dsti.at[r - 2]], ssem).wait()

            compute(rdb, hb, exb, msgb)
            pltpu.async_copy(msgb, agg_sh.at[dsti.at[r]], ssem, add=True)

            @pl.when(c < NCH // 2 - 1)
            def _():
                issue_in(r + 2, rdb, hb, exb, gsem, lsem)

        def chunk2(c, _):
            half(c, 2 * c, rdb0, hb0, exb0, msgb0, gsem0, lsem0, ssem0)
            half(c, 2 * c + 1, rdb1, hb1, exb1, msgb1, gsem1, lsem1, ssem1)
            return 0

        lax.fori_loop(0, NCH // 2, chunk2, 0)
        pltpu.make_async_copy(
            msgb0, agg_sh.at[dsti.at[NCH - 2]], ssem0).wait()
        pltpu.make_async_copy(
            msgb1, agg_sh.at[dsti.at[NCH - 1]], ssem1).wait()
        plsc.subcore_barrier()
        pltpu.sync_copy(
            agg_sh.at[pl.ds(sid * NROWS, NROWS)],
            agg_out.at[cid, pl.ds(sid * NROWS, NROWS), :])

    return k(ex, rden, h, src2d, dst2d)


def _conv_segops(ss, sd, et, k32, h, srcp, dstp):
    dparts, ex = _sc_phase_a(ss, sd, et, k32, srcp, dstp)
    rden = _rden(dparts)
    return _sc_phase_b(ex, rden, h, srcp, dstp)


def kernel(x, edge_index, edge_attr, y, adj, W1, a_src1, a_dst1, a_edge1,
           Wp1, a_src_p1, a_dst_p1, a_edge_p1, Wp2, a_src_p2, a_dst_p2,
           a_edge_p2, Wf1, bf1, Wf2, bf2):
    n, dfeat = x.shape
    e = edge_index.shape[1]

    xp_in = jnp.pad(x, ((0, N_PAD - n), (0, 0)))
    srcp = jnp.concatenate(
        [edge_index[0].astype(jnp.int32),
         jnp.zeros((E_PAD - e,), jnp.int32)]).reshape(E_PAD // 128, 128)
    dstp = jnp.concatenate(
        [edge_index[1].astype(jnp.int32),
         jnp.full((E_PAD - e,), n, jnp.int32)]).reshape(E_PAD // 128, 128)
    eap = jnp.pad(edge_attr, ((0, E_PAD - e), (0, 0)))

    w1p = jnp.pad(W1, ((0, 0), (0, 2)))
    asrc1 = jnp.zeros((32, 32), jnp.float32)
    adst1 = jnp.zeros((32, 32), jnp.float32)
    ae1p = jnp.zeros((4, 32), jnp.float32)
    for hh in range(5):
        blk_s = jnp.broadcast_to(a_src1[hh][:, None], (6, 6))
        blk_d = jnp.broadcast_to(a_dst1[hh][:, None], (6, 6))
        asrc1 = asrc1.at[hh * 6:(hh + 1) * 6, hh * 6:(hh + 1) * 6].set(blk_s)
        adst1 = adst1.at[hh * 6:(hh + 1) * 6, hh * 6:(hh + 1) * 6].set(blk_d)
        ae1p = ae1p.at[:, hh * 6:(hh + 1) * 6].set(
            jnp.broadcast_to(a_edge1[:, hh:hh + 1], (4, 6)))

    wp1p = jnp.pad(Wp1, ((0, 2), (0, 0)))
    asrc2 = jnp.broadcast_to(a_src_p1[0][:, None], (32, 32))
    adst2 = jnp.broadcast_to(a_dst_p1[0][:, None], (32, 32))
    ae2p = jnp.broadcast_to(a_edge_p1, (4, 32))

    h1p, ss1, sd1, mnode1 = _node_prep(xp_in, w1p, asrc1, adst1)
    et1, et2, met1, met2 = _edge_prep(eap, ae1p, ae2p)
    m1 = mnode1[0] + met1[0]
    k1 = jnp.maximum(m1, 0.2 * m1)

    aggparts1 = _conv_segops(ss1, sd1, et1, k1, h1p, srcp, dstp)

    x1, h2p, ss2, sd2, mnode2 = _prep2(aggparts1, wp1p, asrc2, adst2)
    m2 = mnode2[0] + met2[0]
    k2 = jnp.maximum(m2, 0.2 * m2)

    aggparts2 = _conv_segops(ss2, sd2, et2, k2, h2p, srcp, dstp)

    s, ent_sum = _smax(aggparts2, n)
    adjp, xp, sts, fro = _adj_pass(adj, s, x1)

    wp2p = jnp.pad(Wp2, ((0, 2), (0, 4)))
    asp2 = jnp.zeros((8, 8), jnp.float32).at[0:4, 0].set(a_src_p2[0])
    adp2 = jnp.zeros((8, 8), jnp.float32).at[0:4, 0].set(a_dst_p2[0])
    aep2 = a_edge_p2.reshape(1, 1)
    wf1g = jnp.zeros((8, 32, 32), jnp.float32).at[0:4, 0:30, :].set(
        Wf1.reshape(4, 30, 32))
    bf1p = bf1.reshape(1, 32)
    wf2p = jnp.zeros((32, 128), jnp.float32).at[:, 0:2].set(Wf2)
    bf2p = jnp.zeros((1, 128), jnp.float32).at[0, 0:2].set(bf2)

    outa, outb = _tail(adjp, xp, sts, fro, ent_sum, wp2p, asp2, adp2, aep2,
                       wf1g, bf1p, wf2p, bf2p)
    return outa[0:1, 0:2], outb[0, 0]
